# Initial kernel scaffold; baseline (speedup 1.0000x reference)
import jax
import jax.numpy as jnp
from jax.experimental import pallas as pl


def kernel(x, edge_index, edge_attr, params):
    raise NotImplementedError("write your pallas kernel here")


# speedup vs baseline: 1.4684x; 1.4684x over previous
import dataclasses
import functools

import jax
import jax.numpy as jnp
from jax import lax
from jax.experimental import pallas as pl
from jax.experimental.pallas import tpu as pltpu
from jax.experimental.pallas import tpu_sc as plsc

N = 10000
NP = 10240
E = 320000
U = 64
F = 128
DEPTH = 3
NB = 2048
EB = 2560
W_GATH = 128
SEG_CH = 3200
EPS = 1e-5

f32 = jnp.float32
i32 = jnp.int32


def _eye(n):
    r = lax.broadcasted_iota(i32, (n, n), 0)
    c = lax.broadcasted_iota(i32, (n, n), 1)
    return jnp.where(r == c, 1.0, 0.0).astype(f32)


def _silu(v):
    return v * jax.nn.sigmoid(v)


def _dotT(a, b):
    return lax.dot_general(a, b, (((1,), (1,)), ((), ())),
                           preferred_element_type=f32)


def _pre_x_body(x_ref, w_ref, b_ref, o_ref):
    pid = pl.program_id(0)
    xb = x_ref[...]
    rows = pid * NB + lax.broadcasted_iota(i32, (NB, U), 0)
    v = _dotT(xb, w_ref[...]) + b_ref[...]
    o_ref[...] = jnp.where(rows < N, _silu(v), 0.0)


def _pre_x(xp, w0, b0r):
    return pl.pallas_call(
        _pre_x_body,
        grid=(NP // NB,),
        in_specs=[
            pl.BlockSpec((NB, F), lambda i: (i, 0)),
            pl.BlockSpec((U, F), lambda i: (0, 0)),
            pl.BlockSpec((1, U), lambda i: (0, 0)),
        ],
        out_specs=pl.BlockSpec((NB, U), lambda i: (i, 0)),
        out_shape=jax.ShapeDtypeStruct((NP, U), f32),
    )(xp, w0, b0r)


def _pre_w_body(ea_ref, wt_ref, b_ref, o_ref):
    ea = ea_ref[...]
    wt = wt_ref[...]
    v = ea[:, 0:1] * wt[0:1, :] + ea[:, 1:2] * wt[1:2, :] + b_ref[...]
    o_ref[...] = _silu(v)


def _pre_w(ea, wet, ber):
    return pl.pallas_call(
        _pre_w_body,
        grid=(E // EB,),
        in_specs=[
            pl.BlockSpec((EB, 2), lambda i: (i, 0)),
            pl.BlockSpec((2, U), lambda i: (0, 0)),
            pl.BlockSpec((1, U), lambda i: (0, 0)),
        ],
        out_specs=pl.BlockSpec((EB, U), lambda i: (i, 0)),
        out_shape=jax.ShapeDtypeStruct((E, U), f32),
    )(ea, wet, ber)


def _node_mm_body(x_ref, w1, b1, w2, b2t, w3, b3, w4, b4,
                  x1_ref, x3_ref, x4_ref, x2t_ref):
    pid = pl.program_id(0)
    xb = x_ref[...]
    rows = pid * NB + lax.broadcasted_iota(i32, (NB, U), 0)
    rmask = rows < N

    def lin(wr, br):
        return jnp.where(rmask, _dotT(xb, wr[...]) + br[...], 0.0)

    x1_ref[...] = lin(w1, b1)
    x3_ref[...] = lin(w3, b3)
    x4_ref[...] = lin(w4, b4)
    cols = pid * NB + lax.broadcasted_iota(i32, (U, NB), 1)
    x2t = _dotT(w2[...], xb) + b2t[...]
    x2t_ref[...] = jnp.where(cols < N, x2t, 0.0)


def _node_mm(x0, w1, b1, w2, b2t, w3, b3, w4, b4):
    full = lambda i: (0, 0)
    return pl.pallas_call(
        _node_mm_body,
        grid=(NP // NB,),
        in_specs=[
            pl.BlockSpec((NB, U), lambda i: (i, 0)),
            pl.BlockSpec((U, U), full), pl.BlockSpec((1, U), full),
            pl.BlockSpec((U, U), full), pl.BlockSpec((U, 1), full),
            pl.BlockSpec((U, U), full), pl.BlockSpec((1, U), full),
            pl.BlockSpec((U, U), full), pl.BlockSpec((1, U), full),
        ],
        out_specs=[
            pl.BlockSpec((NB, U), lambda i: (i, 0)),
            pl.BlockSpec((NB, U), lambda i: (i, 0)),
            pl.BlockSpec((NB, U), lambda i: (i, 0)),
            pl.BlockSpec((U, NB), lambda i: (0, i)),
        ],
        out_shape=[
            jax.ShapeDtypeStruct((NP, U), f32),
            jax.ShapeDtypeStruct((NP, U), f32),
            jax.ShapeDtypeStruct((NP, U), f32),
            jax.ShapeDtypeStruct((U, NP), f32),
        ],
    )(x0, w1, b1, w2, b2t, w3, b3, w4, b4)


def _edge_mm_body(w_ref, we, be, w1_ref, w2t_ref):
    wb = w_ref[...]
    w1_ref[...] = _dotT(wb, we[...]) + be[...]
    wbt = _dotT(_eye(U), wb)
    w2t_ref[...] = jax.nn.sigmoid(wbt)


def _edge_mm(w0, we, ber):
    full = lambda i: (0, 0)
    return pl.pallas_call(
        _edge_mm_body,
        grid=(E // EB,),
        in_specs=[
            pl.BlockSpec((EB, U), lambda i: (i, 0)),
            pl.BlockSpec((U, U), full), pl.BlockSpec((1, U), full),
        ],
        out_specs=[
            pl.BlockSpec((EB, U), lambda i: (i, 0)),
            pl.BlockSpec((U, EB), lambda i: (0, i)),
        ],
        out_shape=[
            jax.ShapeDtypeStruct((E, U), f32),
            jax.ShapeDtypeStruct((U, E), f32),
        ],
    )(w0, we, ber)


def _stats_body(nrows, a_ref, b_ref, o_ref, st_ref):
    t = a_ref[...] + b_ref[...]
    o_ref[...] = t
    s = jnp.sum(t, axis=0, keepdims=True)
    q = jnp.sum(t * t, axis=0, keepdims=True)
    blk = jnp.concatenate([s, q, jnp.zeros((6, U), f32)], axis=0)

    @pl.when(pl.program_id(0) == 0)
    def _():
        st_ref[...] = blk

    @pl.when(pl.program_id(0) != 0)
    def _():
        st_ref[...] = st_ref[...] + blk


def _edge_stats(w1, g34):
    return pl.pallas_call(
        functools.partial(_stats_body, E),
        grid=(E // EB,),
        in_specs=[
            pl.BlockSpec((EB, U), lambda i: (i, 0)),
            pl.BlockSpec((EB, U), lambda i: (i, 0)),
        ],
        out_specs=[
            pl.BlockSpec((EB, U), lambda i: (i, 0)),
            pl.BlockSpec((8, U), lambda i: (0, 0)),
        ],
        out_shape=[
            jax.ShapeDtypeStruct((E, U), f32),
            jax.ShapeDtypeStruct((8, U), f32),
        ],
    )(w1, g34)


def _node_stats_body(x1_ref, aggt_ref, o_ref, st_ref):
    t = x1_ref[...] + lax.dot_general(aggt_ref[...], _eye(U),
                                      (((0,), (0,)), ((), ())),
                                      preferred_element_type=f32)
    o_ref[...] = t
    s = jnp.sum(t, axis=0, keepdims=True)
    q = jnp.sum(t * t, axis=0, keepdims=True)
    blk = jnp.concatenate([s, q, jnp.zeros((6, U), f32)], axis=0)

    @pl.when(pl.program_id(0) == 0)
    def _():
        st_ref[...] = blk

    @pl.when(pl.program_id(0) != 0)
    def _():
        st_ref[...] = st_ref[...] + blk


def _node_stats(x1, aggt):
    return pl.pallas_call(
        _node_stats_body,
        grid=(NP // NB,),
        in_specs=[
            pl.BlockSpec((NB, U), lambda i: (i, 0)),
            pl.BlockSpec((U, NB), lambda i: (0, i)),
        ],
        out_specs=[
            pl.BlockSpec((NB, U), lambda i: (i, 0)),
            pl.BlockSpec((8, U), lambda i: (0, 0)),
        ],
        out_shape=[
            jax.ShapeDtypeStruct((NP, U), f32),
            jax.ShapeDtypeStruct((8, U), f32),
        ],
    )(x1, aggt)


def _fin_body(count, x0_ref, t_ref, st_ref, g_ref, b_ref, o_ref):
    st = st_ref[...]
    mean = st[0:1, :] / count
    var = st[1:2, :] / count - mean * mean
    istd = lax.rsqrt(var + EPS)
    t = t_ref[...]
    bn = (t - mean) * istd * g_ref[...] + b_ref[...]
    o_ref[...] = x0_ref[...] + _silu(bn)


def _finalize(x0, t, st, gam, bet, rows, blk):
    return pl.pallas_call(
        functools.partial(_fin_body, float(E if rows == E else N)),
        grid=(rows // blk,),
        in_specs=[
            pl.BlockSpec((blk, U), lambda i: (i, 0)),
            pl.BlockSpec((blk, U), lambda i: (i, 0)),
            pl.BlockSpec((8, U), lambda i: (0, 0)),
            pl.BlockSpec((1, U), lambda i: (0, 0)),
            pl.BlockSpec((1, U), lambda i: (0, 0)),
        ],
        out_specs=pl.BlockSpec((blk, U), lambda i: (i, 0)),
        out_shape=jax.ShapeDtypeStruct((rows, U), f32),
    )(x0, t, st, gam, bet)


_MESH = None


def _mesh():
    global _MESH
    if _MESH is None:
        _MESH = plsc.VectorSubcoreMesh(core_axis_name="c",
                                       subcore_axis_name="s")
    return _MESH


def _gather_add(x3, x4, src2d, dst2d):
    @functools.partial(
        pl.kernel,
        out_type=jax.ShapeDtypeStruct((E, U), f32),
        mesh=_mesh(),
        compiler_params=_sc_params(tc_tiling=False),
        scratch_types=[pltpu.VMEM((W_GATH, U), f32)],
    )
    def k(x3_hbm, x4_hbm, src_hbm, dst_hbm, o_hbm, buf4):
        def body(s_v, d_v, o_v):
            pltpu.sync_copy(x3_hbm.at[s_v.at[0]], o_v)
            pltpu.sync_copy(x4_hbm.at[d_v.at[0]], buf4)

            @pl.loop(0, W_GATH)
            def _(r):
                for c in range(0, U, 16):
                    sl = (r, pl.ds(c, 16))
                    o_v[sl] = o_v[sl] + buf4[sl]

        pltpu.emit_pipeline(
            body,
            grid=(E // W_GATH,),
            in_specs=[
                pl.BlockSpec((1, W_GATH), lambda i: (0, i)),
                pl.BlockSpec((1, W_GATH), lambda i: (0, i)),
            ],
            out_specs=[pl.BlockSpec((W_GATH, U), lambda i: (i, 0))],
            core_axis_name=("c", "s"),
            dimension_semantics=(pltpu.PARALLEL,),
        )(src_hbm, dst_hbm, o_hbm)

    return k(x3, x4, src2d, dst2d)


def _sc_params(tc_tiling=True):
    cp = pltpu.CompilerParams()
    if "needs_layout_passes" in pltpu.CompilerParams.__dataclass_fields__:
        cp = dataclasses.replace(cp, needs_layout_passes=False)
    if not tc_tiling:
        cp = dataclasses.replace(cp, use_tc_tiling_on_sc=False)
    return cp


def _segmax(x2t, w2t, src, dst):
    NCH = E // SEG_CH
    GR = SEG_CH // 16

    @functools.partial(
        pl.kernel,
        out_type=jax.ShapeDtypeStruct((U, NP), f32),
        mesh=_mesh(),
        compiler_params=_sc_params(),
        scratch_types=[
            pltpu.VMEM((2, NP), f32),
            pltpu.VMEM((2, NP), f32),
            pltpu.VMEM((NP,), i32),
            pltpu.VMEM((2, SEG_CH), f32),
            pltpu.VMEM((2, SEG_CH), f32),
            pltpu.VMEM((SEG_CH,), i32),
            pltpu.VMEM((SEG_CH,), i32),
            pltpu.VMEM((SEG_CH,), i32),
            pltpu.VMEM((SEG_CH,), i32),
            pltpu.SemaphoreType.DMA,
            pltpu.SemaphoreType.DMA,
            pltpu.SemaphoreType.DMA,
        ],
    )
    def k(x2t_hbm, w2t_hbm, src_hbm, dst_hbm, agg_hbm,
          x2c, acc, scr, w2a, w2b, sa, sb, da, db, semA, semB, semC):
        cid = lax.axis_index("c")
        sid = lax.axis_index("s")
        wid = sid * 2 + cid
        c0 = wid * 2

        pltpu.async_copy(x2t_hbm.at[pl.ds(c0, 2)], x2c, semC).wait()

        neg = jnp.full((16,), -jnp.inf, f32)

        @pl.loop(0, NP // 16)
        def _(i):
            acc[0, pl.ds(i * 16, 16)] = neg
            acc[1, pl.ds(i * 16, 16)] = neg

        def start(ch, w2buf, sbuf, dbuf, sem):
            e0 = ch * SEG_CH
            pltpu.async_copy(
                w2t_hbm.at[pl.ds(c0, 2), pl.ds(e0, SEG_CH)], w2buf, sem)
            pltpu.async_copy(src_hbm.at[pl.ds(e0, SEG_CH)], sbuf, sem)
            pltpu.async_copy(dst_hbm.at[pl.ds(e0, SEG_CH)], dbuf, sem)

        def wait(w2buf, sbuf, dbuf, sem):
            pltpu.make_async_copy(
                w2t_hbm.at[pl.ds(0, 2), pl.ds(0, SEG_CH)], w2buf, sem).wait()
            pltpu.make_async_copy(src_hbm.at[pl.ds(0, SEG_CH)], sbuf,
                                  sem).wait()
            pltpu.make_async_copy(dst_hbm.at[pl.ds(0, SEG_CH)], dbuf,
                                  sem).wait()

        z16 = jnp.zeros((16,), i32)
        o16 = jnp.ones((16,), i32)
        lane = lax.iota(i32, 16)

        def process(w2buf, sbuf, dbuf):
            @pl.loop(0, GR)
            def _(g):
                b = g * 16
                s = sbuf[pl.ds(b, 16)]
                d = dbuf[pl.ds(b, 16)]
                xv0 = plsc.load_gather(x2c, [z16, d])
                xv1 = plsc.load_gather(x2c, [o16, d])
                m0 = w2buf[0, pl.ds(b, 16)] * xv0
                m1 = w2buf[1, pl.ds(b, 16)] * xv1

                def cond(a):
                    return jnp.any(a)

                def body(a):
                    plsc.store_scatter(scr, [s], lane, mask=a)
                    r = plsc.load_gather(scr, [s])
                    win = jnp.logical_and(a, r == lane)
                    a0 = plsc.load_gather(acc, [z16, s])
                    plsc.store_scatter(acc, [z16, s],
                                       jnp.maximum(a0, m0), mask=win)
                    a1 = plsc.load_gather(acc, [o16, s])
                    plsc.store_scatter(acc, [o16, s],
                                       jnp.maximum(a1, m1), mask=win)
                    return jnp.logical_and(a, jnp.logical_not(win))

                lax.while_loop(cond, body, s >= 0)

        start(0, w2a, sa, da, semA)
        start(1, w2b, sb, db, semB)

        @pl.loop(0, NCH, step=2)
        def _(ch):
            wait(w2a, sa, da, semA)
            process(w2a, sa, da)

            @pl.when(ch + 2 < NCH)
            def _():
                start(ch + 2, w2a, sa, da, semA)

            wait(w2b, sb, db, semB)
            process(w2b, sb, db)

            @pl.when(ch + 3 < NCH)
            def _():
                start(ch + 3, w2b, sb, db, semB)

        ninf = jnp.full((16,), -jnp.inf, f32)

        @pl.loop(0, NP // 16)
        def _(i):
            sl0 = (0, pl.ds(i * 16, 16))
            sl1 = (1, pl.ds(i * 16, 16))
            v0 = acc[sl0]
            v1 = acc[sl1]
            acc[sl0] = jnp.where(v0 == ninf, 0.0, v0)
            acc[sl1] = jnp.where(v1 == ninf, 0.0, v1)

        pltpu.async_copy(acc, agg_hbm.at[pl.ds(c0, 2)], semC).wait()

    return k(x2t, w2t, src, dst)


def kernel(x, edge_index, edge_attr, params):
    src = edge_index[0].astype(i32)
    dst = edge_index[1].astype(i32)
    src2d = src.reshape(1, E)
    dst2d = dst.reshape(1, E)

    w0v, b0v = params['v_lin0']
    w0e, b0e = params['e_lin0']

    xp = jnp.pad(x, ((0, NP - N), (0, 0)))
    xc = _pre_x(xp, w0v, b0v.reshape(1, U))
    wc = _pre_w(edge_attr, w0e.T, b0e.reshape(1, U))

    for i in range(DEPTH):
        w1v, b1v = params['v_lins1'][i]
        w2v, b2v = params['v_lins2'][i]
        w3v, b3v = params['v_lins3'][i]
        w4v, b4v = params['v_lins4'][i]
        wev, bev = params['e_lins0'][i]
        gn, bn_ = params['v_bns'][i]
        ge, be_ = params['e_bns'][i]

        x1, x3, x4, x2t = _node_mm(
            xc, w1v, b1v.reshape(1, U), w2v, b2v.reshape(U, 1),
            w3v, b3v.reshape(1, U), w4v, b4v.reshape(1, U))
        w1, w2t = _edge_mm(wc, wev, bev.reshape(1, U))
        g34 = _gather_add(x3, x4, src2d, dst2d)
        aggt = _segmax(x2t, w2t, src, dst)
        te, ste = _edge_stats(w1, g34)
        tn, stn = _node_stats(x1, aggt)
        xc = _finalize(xc, tn, stn, gn.reshape(1, U), bn_.reshape(1, U),
                       NP, NB)
        wc = _finalize(wc, te, ste, ge.reshape(1, U), be_.reshape(1, U),
                       E, EB)

    return xc[:N], wc


# speedup vs baseline: 1.7416x; 1.1860x over previous
import dataclasses
import functools

import jax
import jax.numpy as jnp
from jax import lax
from jax.experimental import pallas as pl
from jax.experimental.pallas import tpu as pltpu
from jax.experimental.pallas import tpu_sc as plsc

N = 10000
NP = 10240
E = 320000
U = 64
F = 128
DEPTH = 3
NB = 2048
EB = 2560
W_GATH = 128
SEG_CH = 3200
EPS = 1e-5

f32 = jnp.float32
i32 = jnp.int32


_GDN = lax.GatherDimensionNumbers(offset_dims=(), collapsed_slice_dims=(0,),
                                  start_index_map=(0,))


def _vtake(v, idx):
    return lax.gather(v, idx[:, None], _GDN, (1,),
                      mode=lax.GatherScatterMode.PROMISE_IN_BOUNDS)


def _eye(n):
    r = lax.broadcasted_iota(i32, (n, n), 0)
    c = lax.broadcasted_iota(i32, (n, n), 1)
    return jnp.where(r == c, 1.0, 0.0).astype(f32)


def _silu(v):
    return v * jax.nn.sigmoid(v)


def _dotT(a, b):
    return lax.dot_general(a, b, (((1,), (1,)), ((), ())),
                           preferred_element_type=f32)


def _pre_x_body(x_ref, w_ref, b_ref, o_ref):
    pid = pl.program_id(0)
    xb = x_ref[...]
    rows = pid * NB + lax.broadcasted_iota(i32, (NB, U), 0)
    v = _dotT(xb, w_ref[...]) + b_ref[...]
    o_ref[...] = jnp.where(rows < N, _silu(v), 0.0)


def _pre_x(xp, w0, b0r):
    return pl.pallas_call(
        _pre_x_body,
        grid=(NP // NB,),
        in_specs=[
            pl.BlockSpec((NB, F), lambda i: (i, 0)),
            pl.BlockSpec((U, F), lambda i: (0, 0)),
            pl.BlockSpec((1, U), lambda i: (0, 0)),
        ],
        out_specs=pl.BlockSpec((NB, U), lambda i: (i, 0)),
        out_shape=jax.ShapeDtypeStruct((NP, U), f32),
    )(xp, w0, b0r)


def _pre_w_body(ea_ref, wt_ref, b_ref, o_ref):
    ea = ea_ref[...]
    wt = wt_ref[...]
    v = ea[:, 0:1] * wt[0:1, :] + ea[:, 1:2] * wt[1:2, :] + b_ref[...]
    o_ref[...] = _silu(v)


def _pre_w(ea, wet, ber):
    return pl.pallas_call(
        _pre_w_body,
        grid=(E // EB,),
        in_specs=[
            pl.BlockSpec((EB, 2), lambda i: (i, 0)),
            pl.BlockSpec((2, U), lambda i: (0, 0)),
            pl.BlockSpec((1, U), lambda i: (0, 0)),
        ],
        out_specs=pl.BlockSpec((EB, U), lambda i: (i, 0)),
        out_shape=jax.ShapeDtypeStruct((E, U), f32),
    )(ea, wet, ber)


def _node_mm_body(x_ref, w1, b1, w2, b2t, w3, b3, w4, b4,
                  x1_ref, x3_ref, x4_ref, x2t_ref):
    pid = pl.program_id(0)
    xb = x_ref[...]
    rows = pid * NB + lax.broadcasted_iota(i32, (NB, U), 0)
    rmask = rows < N

    def lin(wr, br):
        return jnp.where(rmask, _dotT(xb, wr[...]) + br[...], 0.0)

    x1_ref[...] = lin(w1, b1)
    x3_ref[...] = lin(w3, b3)
    x4_ref[...] = lin(w4, b4)
    cols = pid * NB + lax.broadcasted_iota(i32, (U, NB), 1)
    x2t = _dotT(w2[...], xb) + b2t[...]
    x2t_ref[...] = jnp.where(cols < N, x2t, 0.0)


def _node_mm(x0, w1, b1, w2, b2t, w3, b3, w4, b4):
    full = lambda i: (0, 0)
    return pl.pallas_call(
        _node_mm_body,
        grid=(NP // NB,),
        in_specs=[
            pl.BlockSpec((NB, U), lambda i: (i, 0)),
            pl.BlockSpec((U, U), full), pl.BlockSpec((1, U), full),
            pl.BlockSpec((U, U), full), pl.BlockSpec((U, 1), full),
            pl.BlockSpec((U, U), full), pl.BlockSpec((1, U), full),
            pl.BlockSpec((U, U), full), pl.BlockSpec((1, U), full),
        ],
        out_specs=[
            pl.BlockSpec((NB, U), lambda i: (i, 0)),
            pl.BlockSpec((NB, U), lambda i: (i, 0)),
            pl.BlockSpec((NB, U), lambda i: (i, 0)),
            pl.BlockSpec((U, NB), lambda i: (0, i)),
        ],
        out_shape=[
            jax.ShapeDtypeStruct((NP, U), f32),
            jax.ShapeDtypeStruct((NP, U), f32),
            jax.ShapeDtypeStruct((NP, U), f32),
            jax.ShapeDtypeStruct((U, NP), f32),
        ],
    )(x0, w1, b1, w2, b2t, w3, b3, w4, b4)


def _edge_mm_body(w_ref, we, be, w1_ref, w2t_ref):
    wb = w_ref[...]
    w1_ref[...] = _dotT(wb, we[...]) + be[...]
    wbt = _dotT(_eye(U), wb)
    w2t_ref[...] = jax.nn.sigmoid(wbt)


def _edge_mm(w0, we, ber):
    full = lambda i: (0, 0)
    return pl.pallas_call(
        _edge_mm_body,
        grid=(E // EB,),
        in_specs=[
            pl.BlockSpec((EB, U), lambda i: (i, 0)),
            pl.BlockSpec((U, U), full), pl.BlockSpec((1, U), full),
        ],
        out_specs=[
            pl.BlockSpec((EB, U), lambda i: (i, 0)),
            pl.BlockSpec((U, EB), lambda i: (0, i)),
        ],
        out_shape=[
            jax.ShapeDtypeStruct((E, U), f32),
            jax.ShapeDtypeStruct((U, E), f32),
        ],
    )(w0, we, ber)


def _stats_body(nrows, a_ref, b_ref, c_ref, o_ref, st_ref):
    t = a_ref[...] + b_ref[...] + c_ref[...]
    o_ref[...] = t
    s = jnp.sum(t, axis=0, keepdims=True)
    q = jnp.sum(t * t, axis=0, keepdims=True)
    blk = jnp.concatenate([s, q, jnp.zeros((6, U), f32)], axis=0)

    @pl.when(pl.program_id(0) == 0)
    def _():
        st_ref[...] = blk

    @pl.when(pl.program_id(0) != 0)
    def _():
        st_ref[...] = st_ref[...] + blk


def _edge_stats(w1, g3, g4):
    return pl.pallas_call(
        functools.partial(_stats_body, E),
        grid=(E // EB,),
        in_specs=[
            pl.BlockSpec((EB, U), lambda i: (i, 0)),
            pl.BlockSpec((EB, U), lambda i: (i, 0)),
            pl.BlockSpec((EB, U), lambda i: (i, 0)),
        ],
        out_specs=[
            pl.BlockSpec((EB, U), lambda i: (i, 0)),
            pl.BlockSpec((8, U), lambda i: (0, 0)),
        ],
        out_shape=[
            jax.ShapeDtypeStruct((E, U), f32),
            jax.ShapeDtypeStruct((8, U), f32),
        ],
    )(w1, g3, g4)


def _node_stats_body(x1_ref, aggt_ref, o_ref, st_ref):
    t = x1_ref[...] + lax.dot_general(aggt_ref[...], _eye(U),
                                      (((0,), (0,)), ((), ())),
                                      preferred_element_type=f32)
    o_ref[...] = t
    s = jnp.sum(t, axis=0, keepdims=True)
    q = jnp.sum(t * t, axis=0, keepdims=True)
    blk = jnp.concatenate([s, q, jnp.zeros((6, U), f32)], axis=0)

    @pl.when(pl.program_id(0) == 0)
    def _():
        st_ref[...] = blk

    @pl.when(pl.program_id(0) != 0)
    def _():
        st_ref[...] = st_ref[...] + blk


def _node_stats(x1, aggt):
    return pl.pallas_call(
        _node_stats_body,
        grid=(NP // NB,),
        in_specs=[
            pl.BlockSpec((NB, U), lambda i: (i, 0)),
            pl.BlockSpec((U, NB), lambda i: (0, i)),
        ],
        out_specs=[
            pl.BlockSpec((NB, U), lambda i: (i, 0)),
            pl.BlockSpec((8, U), lambda i: (0, 0)),
        ],
        out_shape=[
            jax.ShapeDtypeStruct((NP, U), f32),
            jax.ShapeDtypeStruct((8, U), f32),
        ],
    )(x1, aggt)


def _fin_body(count, x0_ref, t_ref, st_ref, g_ref, b_ref, o_ref):
    st = st_ref[...]
    mean = st[0:1, :] / count
    var = st[1:2, :] / count - mean * mean
    istd = lax.rsqrt(var + EPS)
    t = t_ref[...]
    bn = (t - mean) * istd * g_ref[...] + b_ref[...]
    o_ref[...] = x0_ref[...] + _silu(bn)


def _finalize(x0, t, st, gam, bet, rows, blk):
    return pl.pallas_call(
        functools.partial(_fin_body, float(E if rows == E else N)),
        grid=(rows // blk,),
        in_specs=[
            pl.BlockSpec((blk, U), lambda i: (i, 0)),
            pl.BlockSpec((blk, U), lambda i: (i, 0)),
            pl.BlockSpec((8, U), lambda i: (0, 0)),
            pl.BlockSpec((1, U), lambda i: (0, 0)),
            pl.BlockSpec((1, U), lambda i: (0, 0)),
        ],
        out_specs=pl.BlockSpec((blk, U), lambda i: (i, 0)),
        out_shape=jax.ShapeDtypeStruct((rows, U), f32),
    )(x0, t, st, gam, bet)


_MESH = None


def _mesh():
    global _MESH
    if _MESH is None:
        _MESH = plsc.VectorSubcoreMesh(core_axis_name="c",
                                       subcore_axis_name="s")
    return _MESH


def _gather_add(x3, x4, src2d, dst2d):
    @functools.partial(
        pl.kernel,
        out_type=[jax.ShapeDtypeStruct((E, U), f32),
                  jax.ShapeDtypeStruct((E, U), f32)],
        mesh=_mesh(),
        compiler_params=_sc_params(tc_tiling=False),
    )
    def k(x3_hbm, x4_hbm, src_hbm, dst_hbm, g3_hbm, g4_hbm):
        def body(s_v, d_v, o3_v, o4_v):
            pltpu.sync_copy(x3_hbm.at[s_v.at[0]], o3_v)
            pltpu.sync_copy(x4_hbm.at[d_v.at[0]], o4_v)

        pltpu.emit_pipeline(
            body,
            grid=(E // W_GATH,),
            in_specs=[
                pl.BlockSpec((1, W_GATH), lambda i: (0, i)),
                pl.BlockSpec((1, W_GATH), lambda i: (0, i)),
            ],
            out_specs=[pl.BlockSpec((W_GATH, U), lambda i: (i, 0)),
                       pl.BlockSpec((W_GATH, U), lambda i: (i, 0))],
            core_axis_name=("c", "s"),
            dimension_semantics=(pltpu.PARALLEL,),
        )(src_hbm, dst_hbm, g3_hbm, g4_hbm)

    return k(x3, x4, src2d, dst2d)


def _sc_params(tc_tiling=True):
    cp = pltpu.CompilerParams()
    if "needs_layout_passes" in pltpu.CompilerParams.__dataclass_fields__:
        cp = dataclasses.replace(cp, needs_layout_passes=False)
    if not tc_tiling:
        cp = dataclasses.replace(cp, use_tc_tiling_on_sc=False)
    return cp


def _segmax(x2t, w2t, src, dst):
    NCH = E // SEG_CH
    GR = SEG_CH // 16

    @functools.partial(
        pl.kernel,
        out_type=jax.ShapeDtypeStruct((U, NP), f32),
        mesh=_mesh(),
        compiler_params=_sc_params(),
        scratch_types=[
            pltpu.VMEM((2, NP), f32),
            pltpu.VMEM((2, NP), f32),
            pltpu.VMEM((2, SEG_CH), f32),
            pltpu.VMEM((2, SEG_CH), f32),
            pltpu.VMEM((SEG_CH,), i32),
            pltpu.VMEM((SEG_CH,), i32),
            pltpu.VMEM((SEG_CH,), i32),
            pltpu.VMEM((SEG_CH,), i32),
            pltpu.SemaphoreType.DMA,
            pltpu.SemaphoreType.DMA,
            pltpu.SemaphoreType.DMA,
        ],
    )
    def k(x2t_hbm, w2t_hbm, src_hbm, dst_hbm, agg_hbm,
          x2c, acc, w2a, w2b, sa, sb, da, db, semA, semB, semC):
        cid = lax.axis_index("c")
        sid = lax.axis_index("s")
        wid = sid * 2 + cid
        c0 = wid * 2

        pltpu.async_copy(x2t_hbm.at[pl.ds(c0, 2)], x2c, semC).wait()

        neg = jnp.full((16,), -jnp.inf, f32)

        @pl.loop(0, NP // 16)
        def _(i):
            acc[0, pl.ds(i * 16, 16)] = neg
            acc[1, pl.ds(i * 16, 16)] = neg

        def start(ch, w2buf, sbuf, dbuf, sem):
            e0 = ch * SEG_CH
            pltpu.async_copy(
                w2t_hbm.at[pl.ds(c0, 2), pl.ds(e0, SEG_CH)], w2buf, sem)
            pltpu.async_copy(src_hbm.at[pl.ds(e0, SEG_CH)], sbuf, sem)
            pltpu.async_copy(dst_hbm.at[pl.ds(e0, SEG_CH)], dbuf, sem)

        def wait(w2buf, sbuf, dbuf, sem):
            pltpu.make_async_copy(
                w2t_hbm.at[pl.ds(0, 2), pl.ds(0, SEG_CH)], w2buf, sem).wait()
            pltpu.make_async_copy(src_hbm.at[pl.ds(0, SEG_CH)], sbuf,
                                  sem).wait()
            pltpu.make_async_copy(dst_hbm.at[pl.ds(0, SEG_CH)], dbuf,
                                  sem).wait()

        z16 = jnp.zeros((16,), i32)
        o16 = jnp.ones((16,), i32)
        lane = lax.iota(i32, 16)

        def process(w2buf, sbuf, dbuf):
            @pl.loop(0, GR)
            def _(g):
                b = g * 16
                s = sbuf[pl.ds(b, 16)]
                d = dbuf[pl.ds(b, 16)]
                xv0 = plsc.load_gather(x2c, [z16, d])
                xv1 = plsc.load_gather(x2c, [o16, d])
                m0 = w2buf[0, pl.ds(b, 16)] * xv0
                m1 = w2buf[1, pl.ds(b, 16)] * xv1

                ss, m0s = plsc.sort_key_val(s, m0)
                ss2, m1s = plsc.sort_key_val(s, m1)

                for dd in (1, 2, 4, 8):
                    idx = jnp.maximum(lane - dd, 0)
                    ks = _vtake(ss, idx)
                    take = jnp.logical_and(ks == ss, lane >= dd)
                    p0 = _vtake(m0s, idx)
                    p1 = _vtake(m1s, idx)
                    m0s = jnp.where(take, jnp.maximum(m0s, p0), m0s)
                    m1s = jnp.where(take, jnp.maximum(m1s, p1), m1s)

                nxt = _vtake(ss, jnp.minimum(lane + 1, 15))
                last = jnp.logical_or(lane == 15, ss != nxt)

                a0 = plsc.load_gather(acc, [z16, ss])
                plsc.store_scatter(acc, [z16, ss],
                                   jnp.maximum(a0, m0s), mask=last)
                a1 = plsc.load_gather(acc, [o16, ss2])
                plsc.store_scatter(acc, [o16, ss2],
                                   jnp.maximum(a1, m1s), mask=last)

        start(0, w2a, sa, da, semA)
        start(1, w2b, sb, db, semB)

        @pl.loop(0, NCH, step=2)
        def _(ch):
            wait(w2a, sa, da, semA)
            process(w2a, sa, da)

            @pl.when(ch + 2 < NCH)
            def _():
                start(ch + 2, w2a, sa, da, semA)

            wait(w2b, sb, db, semB)
            process(w2b, sb, db)

            @pl.when(ch + 3 < NCH)
            def _():
                start(ch + 3, w2b, sb, db, semB)

        ninf = jnp.full((16,), -jnp.inf, f32)

        @pl.loop(0, NP // 16)
        def _(i):
            sl0 = (0, pl.ds(i * 16, 16))
            sl1 = (1, pl.ds(i * 16, 16))
            v0 = acc[sl0]
            v1 = acc[sl1]
            acc[sl0] = jnp.where(v0 == ninf, 0.0, v0)
            acc[sl1] = jnp.where(v1 == ninf, 0.0, v1)

        pltpu.async_copy(acc, agg_hbm.at[pl.ds(c0, 2)], semC).wait()

    return k(x2t, w2t, src, dst)


def kernel(x, edge_index, edge_attr, params):
    src = edge_index[0].astype(i32)
    dst = edge_index[1].astype(i32)
    src2d = src.reshape(1, E)
    dst2d = dst.reshape(1, E)

    w0v, b0v = params['v_lin0']
    w0e, b0e = params['e_lin0']

    xp = jnp.pad(x, ((0, NP - N), (0, 0)))
    xc = _pre_x(xp, w0v, b0v.reshape(1, U))
    wc = _pre_w(edge_attr, w0e.T, b0e.reshape(1, U))

    for i in range(DEPTH):
        w1v, b1v = params['v_lins1'][i]
        w2v, b2v = params['v_lins2'][i]
        w3v, b3v = params['v_lins3'][i]
        w4v, b4v = params['v_lins4'][i]
        wev, bev = params['e_lins0'][i]
        gn, bn_ = params['v_bns'][i]
        ge, be_ = params['e_bns'][i]

        x1, x3, x4, x2t = _node_mm(
            xc, w1v, b1v.reshape(1, U), w2v, b2v.reshape(U, 1),
            w3v, b3v.reshape(1, U), w4v, b4v.reshape(1, U))
        w1, w2t = _edge_mm(wc, wev, bev.reshape(1, U))
        g3, g4 = _gather_add(x3, x4, src2d, dst2d)
        aggt = _segmax(x2t, w2t, src, dst)
        te, ste = _edge_stats(w1, g3, g4)
        tn, stn = _node_stats(x1, aggt)
        xc = _finalize(xc, tn, stn, gn.reshape(1, U), bn_.reshape(1, U),
                       NP, NB)
        wc = _finalize(wc, te, ste, ge.reshape(1, U), be_.reshape(1, U),
                       E, EB)

    return xc[:N], wc


# speedup vs baseline: 1.7513x; 1.0056x over previous
import dataclasses
import functools

import jax
import jax.numpy as jnp
from jax import lax
from jax.experimental import pallas as pl
from jax.experimental.pallas import tpu as pltpu
from jax.experimental.pallas import tpu_sc as plsc

N = 10000
NP = 10240
E = 320000
U = 64
F = 128
DEPTH = 3
NB = 2048
EB = 2560
W_GATH = 128
SEG_CH = 3200
EPS = 1e-5

f32 = jnp.float32
i32 = jnp.int32


_GDN = lax.GatherDimensionNumbers(offset_dims=(), collapsed_slice_dims=(0,),
                                  start_index_map=(0,))


def _vtake(v, idx):
    return lax.gather(v, idx[:, None], _GDN, (1,),
                      mode=lax.GatherScatterMode.PROMISE_IN_BOUNDS)


def _eye(n):
    r = lax.broadcasted_iota(i32, (n, n), 0)
    c = lax.broadcasted_iota(i32, (n, n), 1)
    return jnp.where(r == c, 1.0, 0.0).astype(f32)


def _silu(v):
    return v * jax.nn.sigmoid(v)


def _dotT(a, b):
    return lax.dot_general(a, b, (((1,), (1,)), ((), ())),
                           preferred_element_type=f32)


def _pre_x_body(x_ref, w_ref, b_ref, o_ref):
    pid = pl.program_id(0)
    xb = x_ref[...]
    rows = pid * NB + lax.broadcasted_iota(i32, (NB, U), 0)
    v = _dotT(xb, w_ref[...]) + b_ref[...]
    o_ref[...] = jnp.where(rows < N, _silu(v), 0.0)


def _pre_x(xp, w0, b0r):
    return pl.pallas_call(
        _pre_x_body,
        grid=(NP // NB,),
        in_specs=[
            pl.BlockSpec((NB, F), lambda i: (i, 0)),
            pl.BlockSpec((U, F), lambda i: (0, 0)),
            pl.BlockSpec((1, U), lambda i: (0, 0)),
        ],
        out_specs=pl.BlockSpec((NB, U), lambda i: (i, 0)),
        out_shape=jax.ShapeDtypeStruct((NP, U), f32),
    )(xp, w0, b0r)


def _pre_w_body(ea_ref, wt_ref, b_ref, o_ref):
    ea = ea_ref[...]
    wt = wt_ref[...]
    v = ea[:, 0:1] * wt[0:1, :] + ea[:, 1:2] * wt[1:2, :] + b_ref[...]
    o_ref[...] = _silu(v)


def _pre_w(ea, wet, ber):
    return pl.pallas_call(
        _pre_w_body,
        grid=(E // EB,),
        in_specs=[
            pl.BlockSpec((EB, 2), lambda i: (i, 0)),
            pl.BlockSpec((2, U), lambda i: (0, 0)),
            pl.BlockSpec((1, U), lambda i: (0, 0)),
        ],
        out_specs=pl.BlockSpec((EB, U), lambda i: (i, 0)),
        out_shape=jax.ShapeDtypeStruct((E, U), f32),
    )(ea, wet, ber)


def _node_mm_body(x_ref, w1, b1, w2, b2t, w3, b3, w4, b4,
                  x1_ref, x3_ref, x4_ref, x2t_ref):
    pid = pl.program_id(0)
    xb = x_ref[...]
    rows = pid * NB + lax.broadcasted_iota(i32, (NB, U), 0)
    rmask = rows < N

    def lin(wr, br):
        return jnp.where(rmask, _dotT(xb, wr[...]) + br[...], 0.0)

    x1_ref[...] = lin(w1, b1)
    x3_ref[...] = lin(w3, b3)
    x4_ref[...] = lin(w4, b4)
    cols = pid * NB + lax.broadcasted_iota(i32, (U, NB), 1)
    x2t = _dotT(w2[...], xb) + b2t[...]
    x2t_ref[...] = jnp.where(cols < N, x2t, 0.0)


def _node_mm(x0, w1, b1, w2, b2t, w3, b3, w4, b4):
    full = lambda i: (0, 0)
    return pl.pallas_call(
        _node_mm_body,
        grid=(NP // NB,),
        in_specs=[
            pl.BlockSpec((NB, U), lambda i: (i, 0)),
            pl.BlockSpec((U, U), full), pl.BlockSpec((1, U), full),
            pl.BlockSpec((U, U), full), pl.BlockSpec((U, 1), full),
            pl.BlockSpec((U, U), full), pl.BlockSpec((1, U), full),
            pl.BlockSpec((U, U), full), pl.BlockSpec((1, U), full),
        ],
        out_specs=[
            pl.BlockSpec((NB, U), lambda i: (i, 0)),
            pl.BlockSpec((NB, U), lambda i: (i, 0)),
            pl.BlockSpec((NB, U), lambda i: (i, 0)),
            pl.BlockSpec((U, NB), lambda i: (0, i)),
        ],
        out_shape=[
            jax.ShapeDtypeStruct((NP, U), f32),
            jax.ShapeDtypeStruct((NP, U), f32),
            jax.ShapeDtypeStruct((NP, U), f32),
            jax.ShapeDtypeStruct((U, NP), f32),
        ],
    )(x0, w1, b1, w2, b2t, w3, b3, w4, b4)


def _edge_mm_body(w_ref, we, be, w1_ref, w2t_ref):
    wb = w_ref[...]
    w1_ref[...] = _dotT(wb, we[...]) + be[...]
    wbt = _dotT(_eye(U), wb)
    w2t_ref[...] = jax.nn.sigmoid(wbt)


def _edge_mm(w0, we, ber):
    full = lambda i: (0, 0)
    return pl.pallas_call(
        _edge_mm_body,
        grid=(E // EB,),
        in_specs=[
            pl.BlockSpec((EB, U), lambda i: (i, 0)),
            pl.BlockSpec((U, U), full), pl.BlockSpec((1, U), full),
        ],
        out_specs=[
            pl.BlockSpec((EB, U), lambda i: (i, 0)),
            pl.BlockSpec((U, EB), lambda i: (0, i)),
        ],
        out_shape=[
            jax.ShapeDtypeStruct((E, U), f32),
            jax.ShapeDtypeStruct((U, E), f32),
        ],
    )(w0, we, ber)


def _stats_body(nrows, a_ref, b_ref, c_ref, o_ref, st_ref):
    t = a_ref[...] + b_ref[...] + c_ref[...]
    o_ref[...] = t
    s = jnp.sum(t, axis=0, keepdims=True)
    q = jnp.sum(t * t, axis=0, keepdims=True)
    blk = jnp.concatenate([s, q, jnp.zeros((6, U), f32)], axis=0)

    @pl.when(pl.program_id(0) == 0)
    def _():
        st_ref[...] = blk

    @pl.when(pl.program_id(0) != 0)
    def _():
        st_ref[...] = st_ref[...] + blk


def _edge_stats(w1, g3, g4):
    return pl.pallas_call(
        functools.partial(_stats_body, E),
        grid=(E // EB,),
        in_specs=[
            pl.BlockSpec((EB, U), lambda i: (i, 0)),
            pl.BlockSpec((EB, U), lambda i: (i, 0)),
            pl.BlockSpec((EB, U), lambda i: (i, 0)),
        ],
        out_specs=[
            pl.BlockSpec((EB, U), lambda i: (i, 0)),
            pl.BlockSpec((8, U), lambda i: (0, 0)),
        ],
        out_shape=[
            jax.ShapeDtypeStruct((E, U), f32),
            jax.ShapeDtypeStruct((8, U), f32),
        ],
    )(w1, g3, g4)


def _node_stats_body(x1_ref, aggt_ref, o_ref, st_ref):
    t = x1_ref[...] + lax.dot_general(aggt_ref[...], _eye(U),
                                      (((0,), (0,)), ((), ())),
                                      preferred_element_type=f32)
    o_ref[...] = t
    s = jnp.sum(t, axis=0, keepdims=True)
    q = jnp.sum(t * t, axis=0, keepdims=True)
    blk = jnp.concatenate([s, q, jnp.zeros((6, U), f32)], axis=0)

    @pl.when(pl.program_id(0) == 0)
    def _():
        st_ref[...] = blk

    @pl.when(pl.program_id(0) != 0)
    def _():
        st_ref[...] = st_ref[...] + blk


def _node_stats(x1, aggt):
    return pl.pallas_call(
        _node_stats_body,
        grid=(NP // NB,),
        in_specs=[
            pl.BlockSpec((NB, U), lambda i: (i, 0)),
            pl.BlockSpec((U, NB), lambda i: (0, i)),
        ],
        out_specs=[
            pl.BlockSpec((NB, U), lambda i: (i, 0)),
            pl.BlockSpec((8, U), lambda i: (0, 0)),
        ],
        out_shape=[
            jax.ShapeDtypeStruct((NP, U), f32),
            jax.ShapeDtypeStruct((8, U), f32),
        ],
    )(x1, aggt)


def _fin_body(count, x0_ref, t_ref, st_ref, g_ref, b_ref, o_ref):
    st = st_ref[...]
    mean = st[0:1, :] / count
    var = st[1:2, :] / count - mean * mean
    istd = lax.rsqrt(var + EPS)
    t = t_ref[...]
    bn = (t - mean) * istd * g_ref[...] + b_ref[...]
    o_ref[...] = x0_ref[...] + _silu(bn)


def _finalize(x0, t, st, gam, bet, rows, blk):
    return pl.pallas_call(
        functools.partial(_fin_body, float(E if rows == E else N)),
        grid=(rows // blk,),
        in_specs=[
            pl.BlockSpec((blk, U), lambda i: (i, 0)),
            pl.BlockSpec((blk, U), lambda i: (i, 0)),
            pl.BlockSpec((8, U), lambda i: (0, 0)),
            pl.BlockSpec((1, U), lambda i: (0, 0)),
            pl.BlockSpec((1, U), lambda i: (0, 0)),
        ],
        out_specs=pl.BlockSpec((blk, U), lambda i: (i, 0)),
        out_shape=jax.ShapeDtypeStruct((rows, U), f32),
    )(x0, t, st, gam, bet)


_MESH = None


def _mesh():
    global _MESH
    if _MESH is None:
        _MESH = plsc.VectorSubcoreMesh(core_axis_name="c",
                                       subcore_axis_name="s")
    return _MESH


def _gather_add(x3, x4, src2d, dst2d):
    @functools.partial(
        pl.kernel,
        out_type=[jax.ShapeDtypeStruct((E, U), f32),
                  jax.ShapeDtypeStruct((E, U), f32)],
        mesh=_mesh(),
        compiler_params=_sc_params(tc_tiling=False),
        scratch_types=[pltpu.SemaphoreType.DMA, pltpu.SemaphoreType.DMA],
    )
    def k(x3_hbm, x4_hbm, src_hbm, dst_hbm, g3_hbm, g4_hbm, sem3, sem4):
        def body(s_v, d_v, o3_v, o4_v):
            c3 = pltpu.async_copy(x3_hbm.at[s_v.at[0]], o3_v, sem3)
            c4 = pltpu.async_copy(x4_hbm.at[d_v.at[0]], o4_v, sem4)
            c3.wait()
            c4.wait()

        pltpu.emit_pipeline(
            body,
            grid=(E // W_GATH,),
            in_specs=[
                pl.BlockSpec((1, W_GATH), lambda i: (0, i)),
                pl.BlockSpec((1, W_GATH), lambda i: (0, i)),
            ],
            out_specs=[pl.BlockSpec((W_GATH, U), lambda i: (i, 0)),
                       pl.BlockSpec((W_GATH, U), lambda i: (i, 0))],
            core_axis_name=("c", "s"),
            dimension_semantics=(pltpu.PARALLEL,),
        )(src_hbm, dst_hbm, g3_hbm, g4_hbm)

    return k(x3, x4, src2d, dst2d)


def _sc_params(tc_tiling=True):
    cp = pltpu.CompilerParams()
    if "needs_layout_passes" in pltpu.CompilerParams.__dataclass_fields__:
        cp = dataclasses.replace(cp, needs_layout_passes=False)
    if not tc_tiling:
        cp = dataclasses.replace(cp, use_tc_tiling_on_sc=False)
    return cp


def _segmax(x2t, w2t, src, dst):
    NCH = E // SEG_CH
    GR = SEG_CH // 16

    @functools.partial(
        pl.kernel,
        out_type=jax.ShapeDtypeStruct((U, NP), f32),
        mesh=_mesh(),
        compiler_params=_sc_params(),
        scratch_types=[
            pltpu.VMEM((2, NP), f32),
            pltpu.VMEM((2, NP), f32),
            pltpu.VMEM((2, SEG_CH), f32),
            pltpu.VMEM((2, SEG_CH), f32),
            pltpu.VMEM((SEG_CH,), i32),
            pltpu.VMEM((SEG_CH,), i32),
            pltpu.VMEM((SEG_CH,), i32),
            pltpu.VMEM((SEG_CH,), i32),
            pltpu.SemaphoreType.DMA,
            pltpu.SemaphoreType.DMA,
            pltpu.SemaphoreType.DMA,
        ],
    )
    def k(x2t_hbm, w2t_hbm, src_hbm, dst_hbm, agg_hbm,
          x2c, acc, w2a, w2b, sa, sb, da, db, semA, semB, semC):
        cid = lax.axis_index("c")
        sid = lax.axis_index("s")
        wid = sid * 2 + cid
        c0 = wid * 2

        pltpu.async_copy(x2t_hbm.at[pl.ds(c0, 2)], x2c, semC).wait()

        neg = jnp.full((16,), -jnp.inf, f32)

        @pl.loop(0, NP // 16)
        def _(i):
            acc[0, pl.ds(i * 16, 16)] = neg
            acc[1, pl.ds(i * 16, 16)] = neg

        def start(ch, w2buf, sbuf, dbuf, sem):
            e0 = ch * SEG_CH
            pltpu.async_copy(
                w2t_hbm.at[pl.ds(c0, 2), pl.ds(e0, SEG_CH)], w2buf, sem)
            pltpu.async_copy(src_hbm.at[pl.ds(e0, SEG_CH)], sbuf, sem)
            pltpu.async_copy(dst_hbm.at[pl.ds(e0, SEG_CH)], dbuf, sem)

        def wait(w2buf, sbuf, dbuf, sem):
            pltpu.make_async_copy(
                w2t_hbm.at[pl.ds(0, 2), pl.ds(0, SEG_CH)], w2buf, sem).wait()
            pltpu.make_async_copy(src_hbm.at[pl.ds(0, SEG_CH)], sbuf,
                                  sem).wait()
            pltpu.make_async_copy(dst_hbm.at[pl.ds(0, SEG_CH)], dbuf,
                                  sem).wait()

        z16 = jnp.zeros((16,), i32)
        o16 = jnp.ones((16,), i32)
        lane = lax.iota(i32, 16)

        def process(w2buf, sbuf, dbuf):
            def group(b):
                s = sbuf[pl.ds(b, 16)]
                d = dbuf[pl.ds(b, 16)]
                xv0 = plsc.load_gather(x2c, [z16, d])
                xv1 = plsc.load_gather(x2c, [o16, d])
                m0 = w2buf[0, pl.ds(b, 16)] * xv0
                m1 = w2buf[1, pl.ds(b, 16)] * xv1

                ss, m0s = plsc.sort_key_val(s, m0)
                ss2, m1s = plsc.sort_key_val(s, m1)

                for dd in (1, 2, 4, 8):
                    idx = jnp.maximum(lane - dd, 0)
                    ks = _vtake(ss, idx)
                    take = jnp.logical_and(ks == ss, lane >= dd)
                    p0 = _vtake(m0s, idx)
                    p1 = _vtake(m1s, idx)
                    m0s = jnp.where(take, jnp.maximum(m0s, p0), m0s)
                    m1s = jnp.where(take, jnp.maximum(m1s, p1), m1s)

                nxt = _vtake(ss, jnp.minimum(lane + 1, 15))
                last = jnp.logical_or(lane == 15, ss != nxt)

                a0 = plsc.load_gather(acc, [z16, ss])
                plsc.store_scatter(acc, [z16, ss],
                                   jnp.maximum(a0, m0s), mask=last)
                a1 = plsc.load_gather(acc, [o16, ss2])
                plsc.store_scatter(acc, [o16, ss2],
                                   jnp.maximum(a1, m1s), mask=last)

            @pl.loop(0, GR, step=2)
            def _(g):
                group(g * 16)
                group(g * 16 + 16)

        start(0, w2a, sa, da, semA)
        start(1, w2b, sb, db, semB)

        @pl.loop(0, NCH, step=2)
        def _(ch):
            wait(w2a, sa, da, semA)
            process(w2a, sa, da)

            @pl.when(ch + 2 < NCH)
            def _():
                start(ch + 2, w2a, sa, da, semA)

            wait(w2b, sb, db, semB)
            process(w2b, sb, db)

            @pl.when(ch + 3 < NCH)
            def _():
                start(ch + 3, w2b, sb, db, semB)

        ninf = jnp.full((16,), -jnp.inf, f32)

        @pl.loop(0, NP // 16)
        def _(i):
            sl0 = (0, pl.ds(i * 16, 16))
            sl1 = (1, pl.ds(i * 16, 16))
            v0 = acc[sl0]
            v1 = acc[sl1]
            acc[sl0] = jnp.where(v0 == ninf, 0.0, v0)
            acc[sl1] = jnp.where(v1 == ninf, 0.0, v1)

        pltpu.async_copy(acc, agg_hbm.at[pl.ds(c0, 2)], semC).wait()

    return k(x2t, w2t, src, dst)


def kernel(x, edge_index, edge_attr, params):
    src = edge_index[0].astype(i32)
    dst = edge_index[1].astype(i32)
    src2d = src.reshape(1, E)
    dst2d = dst.reshape(1, E)

    w0v, b0v = params['v_lin0']
    w0e, b0e = params['e_lin0']

    xp = jnp.pad(x, ((0, NP - N), (0, 0)))
    xc = _pre_x(xp, w0v, b0v.reshape(1, U))
    wc = _pre_w(edge_attr, w0e.T, b0e.reshape(1, U))

    for i in range(DEPTH):
        w1v, b1v = params['v_lins1'][i]
        w2v, b2v = params['v_lins2'][i]
        w3v, b3v = params['v_lins3'][i]
        w4v, b4v = params['v_lins4'][i]
        wev, bev = params['e_lins0'][i]
        gn, bn_ = params['v_bns'][i]
        ge, be_ = params['e_bns'][i]

        x1, x3, x4, x2t = _node_mm(
            xc, w1v, b1v.reshape(1, U), w2v, b2v.reshape(U, 1),
            w3v, b3v.reshape(1, U), w4v, b4v.reshape(1, U))
        w1, w2t = _edge_mm(wc, wev, bev.reshape(1, U))
        g3, g4 = _gather_add(x3, x4, src2d, dst2d)
        aggt = _segmax(x2t, w2t, src, dst)
        te, ste = _edge_stats(w1, g3, g4)
        tn, stn = _node_stats(x1, aggt)
        xc = _finalize(xc, tn, stn, gn.reshape(1, U), bn_.reshape(1, U),
                       NP, NB)
        wc = _finalize(wc, te, ste, ge.reshape(1, U), be_.reshape(1, U),
                       E, EB)

    return xc[:N], wc


# speedup vs baseline: 1.9128x; 1.0922x over previous
import dataclasses
import functools

import jax
import jax.numpy as jnp
from jax import lax
from jax.experimental import pallas as pl
from jax.experimental.pallas import tpu as pltpu
from jax.experimental.pallas import tpu_sc as plsc

N = 10000
NP = 10240
E = 320000
U = 64
F = 128
DEPTH = 3
NB = 2048
EB = 2560
W_GATH = 128
SEG_CH = 3200
EPS = 1e-5

f32 = jnp.float32
i32 = jnp.int32


_GDN = lax.GatherDimensionNumbers(offset_dims=(), collapsed_slice_dims=(0,),
                                  start_index_map=(0,))


def _vtake(v, idx):
    return lax.gather(v, idx[:, None], _GDN, (1,),
                      mode=lax.GatherScatterMode.PROMISE_IN_BOUNDS)


def _eye(n):
    r = lax.broadcasted_iota(i32, (n, n), 0)
    c = lax.broadcasted_iota(i32, (n, n), 1)
    return jnp.where(r == c, 1.0, 0.0).astype(f32)


def _silu(v):
    return v * jax.nn.sigmoid(v)


def _dotT(a, b):
    return lax.dot_general(a, b, (((1,), (1,)), ((), ())),
                           preferred_element_type=f32)


def _pre_x_body(x_ref, w_ref, b_ref, o_ref):
    pid = pl.program_id(0)
    xb = x_ref[...]
    rows = pid * NB + lax.broadcasted_iota(i32, (NB, U), 0)
    v = _dotT(xb, w_ref[...]) + b_ref[...]
    o_ref[...] = jnp.where(rows < N, _silu(v), 0.0)


def _pre_x(xp, w0, b0r):
    return pl.pallas_call(
        _pre_x_body,
        grid=(NP // NB,),
        in_specs=[
            pl.BlockSpec((NB, F), lambda i: (i, 0)),
            pl.BlockSpec((U, F), lambda i: (0, 0)),
            pl.BlockSpec((1, U), lambda i: (0, 0)),
        ],
        out_specs=pl.BlockSpec((NB, U), lambda i: (i, 0)),
        out_shape=jax.ShapeDtypeStruct((NP, U), f32),
    )(xp, w0, b0r)


def _pre_w_body(ea_ref, wt_ref, b_ref, o_ref):
    ea = ea_ref[...]
    wt = wt_ref[...]
    v = ea[:, 0:1] * wt[0:1, :] + ea[:, 1:2] * wt[1:2, :] + b_ref[...]
    o_ref[...] = _silu(v)


def _pre_w(ea, wet, ber):
    return pl.pallas_call(
        _pre_w_body,
        grid=(E // EB,),
        in_specs=[
            pl.BlockSpec((EB, 2), lambda i: (i, 0)),
            pl.BlockSpec((2, U), lambda i: (0, 0)),
            pl.BlockSpec((1, U), lambda i: (0, 0)),
        ],
        out_specs=pl.BlockSpec((EB, U), lambda i: (i, 0)),
        out_shape=jax.ShapeDtypeStruct((E, U), f32),
    )(ea, wet, ber)


def _node_mm_body(x_ref, w1, b1, w2, b2t, w3, b3, w4, b4,
                  x1_ref, x3_ref, x4_ref, x2t_ref):
    pid = pl.program_id(0)
    xb = x_ref[...]
    rows = pid * NB + lax.broadcasted_iota(i32, (NB, U), 0)
    rmask = rows < N

    def lin(wr, br):
        return jnp.where(rmask, _dotT(xb, wr[...]) + br[...], 0.0)

    x1_ref[...] = lin(w1, b1)
    x3_ref[...] = lin(w3, b3)
    x4_ref[...] = lin(w4, b4)
    cols = pid * NB + lax.broadcasted_iota(i32, (U, NB), 1)
    x2t = _dotT(w2[...], xb) + b2t[...]
    x2t_ref[...] = jnp.where(cols < N, x2t, 0.0)


def _node_mm(x0, w1, b1, w2, b2t, w3, b3, w4, b4):
    full = lambda i: (0, 0)
    return pl.pallas_call(
        _node_mm_body,
        grid=(NP // NB,),
        in_specs=[
            pl.BlockSpec((NB, U), lambda i: (i, 0)),
            pl.BlockSpec((U, U), full), pl.BlockSpec((1, U), full),
            pl.BlockSpec((U, U), full), pl.BlockSpec((U, 1), full),
            pl.BlockSpec((U, U), full), pl.BlockSpec((1, U), full),
            pl.BlockSpec((U, U), full), pl.BlockSpec((1, U), full),
        ],
        out_specs=[
            pl.BlockSpec((NB, U), lambda i: (i, 0)),
            pl.BlockSpec((NB, U), lambda i: (i, 0)),
            pl.BlockSpec((NB, U), lambda i: (i, 0)),
            pl.BlockSpec((U, NB), lambda i: (0, i)),
        ],
        out_shape=[
            jax.ShapeDtypeStruct((NP, U), f32),
            jax.ShapeDtypeStruct((NP, U), f32),
            jax.ShapeDtypeStruct((NP, U), f32),
            jax.ShapeDtypeStruct((U, NP), f32),
        ],
    )(x0, w1, b1, w2, b2t, w3, b3, w4, b4)


def _edge_mm_body(w_ref, we, be, w1_ref, w2t_ref):
    wb = w_ref[...]
    w1_ref[...] = _dotT(wb, we[...]) + be[...]
    wbt = _dotT(_eye(U), wb)
    w2t_ref[...] = jax.nn.sigmoid(wbt)


def _edge_mm(w0, we, ber):
    full = lambda i: (0, 0)
    return pl.pallas_call(
        _edge_mm_body,
        grid=(E // EB,),
        in_specs=[
            pl.BlockSpec((EB, U), lambda i: (i, 0)),
            pl.BlockSpec((U, U), full), pl.BlockSpec((1, U), full),
        ],
        out_specs=[
            pl.BlockSpec((EB, U), lambda i: (i, 0)),
            pl.BlockSpec((U, EB), lambda i: (0, i)),
        ],
        out_shape=[
            jax.ShapeDtypeStruct((E, U), f32),
            jax.ShapeDtypeStruct((U, E), f32),
        ],
    )(w0, we, ber)


def _stats_body(nrows, a_ref, b_ref, c_ref, o_ref, st_ref):
    t = a_ref[...] + b_ref[...] + c_ref[...]
    o_ref[...] = t
    s = jnp.sum(t, axis=0, keepdims=True)
    q = jnp.sum(t * t, axis=0, keepdims=True)
    blk = jnp.concatenate([s, q, jnp.zeros((6, U), f32)], axis=0)

    @pl.when(pl.program_id(0) == 0)
    def _():
        st_ref[...] = blk

    @pl.when(pl.program_id(0) != 0)
    def _():
        st_ref[...] = st_ref[...] + blk


def _edge_stats(w1, g3, g4):
    return pl.pallas_call(
        functools.partial(_stats_body, E),
        grid=(E // EB,),
        in_specs=[
            pl.BlockSpec((EB, U), lambda i: (i, 0)),
            pl.BlockSpec((EB, U), lambda i: (i, 0)),
            pl.BlockSpec((EB, U), lambda i: (i, 0)),
        ],
        out_specs=[
            pl.BlockSpec((EB, U), lambda i: (i, 0)),
            pl.BlockSpec((8, U), lambda i: (0, 0)),
        ],
        out_shape=[
            jax.ShapeDtypeStruct((E, U), f32),
            jax.ShapeDtypeStruct((8, U), f32),
        ],
    )(w1, g3, g4)


def _node_stats_body(x1_ref, aggt_ref, o_ref, st_ref):
    t = x1_ref[...] + lax.dot_general(aggt_ref[...], _eye(U),
                                      (((0,), (0,)), ((), ())),
                                      preferred_element_type=f32)
    o_ref[...] = t
    s = jnp.sum(t, axis=0, keepdims=True)
    q = jnp.sum(t * t, axis=0, keepdims=True)
    blk = jnp.concatenate([s, q, jnp.zeros((6, U), f32)], axis=0)

    @pl.when(pl.program_id(0) == 0)
    def _():
        st_ref[...] = blk

    @pl.when(pl.program_id(0) != 0)
    def _():
        st_ref[...] = st_ref[...] + blk


def _node_stats(x1, aggt):
    return pl.pallas_call(
        _node_stats_body,
        grid=(NP // NB,),
        in_specs=[
            pl.BlockSpec((NB, U), lambda i: (i, 0)),
            pl.BlockSpec((U, NB), lambda i: (0, i)),
        ],
        out_specs=[
            pl.BlockSpec((NB, U), lambda i: (i, 0)),
            pl.BlockSpec((8, U), lambda i: (0, 0)),
        ],
        out_shape=[
            jax.ShapeDtypeStruct((NP, U), f32),
            jax.ShapeDtypeStruct((8, U), f32),
        ],
    )(x1, aggt)


def _fin_body(count, x0_ref, t_ref, st_ref, g_ref, b_ref, o_ref):
    st = st_ref[...]
    mean = st[0:1, :] / count
    var = st[1:2, :] / count - mean * mean
    istd = lax.rsqrt(var + EPS)
    t = t_ref[...]
    bn = (t - mean) * istd * g_ref[...] + b_ref[...]
    o_ref[...] = x0_ref[...] + _silu(bn)


def _finalize(x0, t, st, gam, bet, rows, blk):
    return pl.pallas_call(
        functools.partial(_fin_body, float(E if rows == E else N)),
        grid=(rows // blk,),
        in_specs=[
            pl.BlockSpec((blk, U), lambda i: (i, 0)),
            pl.BlockSpec((blk, U), lambda i: (i, 0)),
            pl.BlockSpec((8, U), lambda i: (0, 0)),
            pl.BlockSpec((1, U), lambda i: (0, 0)),
            pl.BlockSpec((1, U), lambda i: (0, 0)),
        ],
        out_specs=pl.BlockSpec((blk, U), lambda i: (i, 0)),
        out_shape=jax.ShapeDtypeStruct((rows, U), f32),
    )(x0, t, st, gam, bet)


_MESH = None


def _mesh():
    global _MESH
    if _MESH is None:
        _MESH = plsc.VectorSubcoreMesh(core_axis_name="c",
                                       subcore_axis_name="s")
    return _MESH


def _gather_add(x3, x4, src2d, dst2d):
    @functools.partial(
        pl.kernel,
        out_type=[jax.ShapeDtypeStruct((E, U), f32),
                  jax.ShapeDtypeStruct((E, U), f32)],
        mesh=_mesh(),
        compiler_params=_sc_params(tc_tiling=False),
        scratch_types=[pltpu.SemaphoreType.DMA, pltpu.SemaphoreType.DMA],
    )
    def k(x3_hbm, x4_hbm, src_hbm, dst_hbm, g3_hbm, g4_hbm, sem3, sem4):
        def body(s_v, d_v, o3_v, o4_v):
            c3 = pltpu.async_copy(x3_hbm.at[s_v.at[0]], o3_v, sem3)
            c4 = pltpu.async_copy(x4_hbm.at[d_v.at[0]], o4_v, sem4)
            c3.wait()
            c4.wait()

        pltpu.emit_pipeline(
            body,
            grid=(E // W_GATH,),
            in_specs=[
                pl.BlockSpec((1, W_GATH), lambda i: (0, i)),
                pl.BlockSpec((1, W_GATH), lambda i: (0, i)),
            ],
            out_specs=[pl.BlockSpec((W_GATH, U), lambda i: (i, 0)),
                       pl.BlockSpec((W_GATH, U), lambda i: (i, 0))],
            core_axis_name=("c", "s"),
            dimension_semantics=(pltpu.PARALLEL,),
        )(src_hbm, dst_hbm, g3_hbm, g4_hbm)

    return k(x3, x4, src2d, dst2d)


def _sc_params(tc_tiling=True):
    cp = pltpu.CompilerParams()
    if "needs_layout_passes" in pltpu.CompilerParams.__dataclass_fields__:
        cp = dataclasses.replace(cp, needs_layout_passes=False)
    if not tc_tiling:
        cp = dataclasses.replace(cp, use_tc_tiling_on_sc=False)
    return cp


def _segmax(x2t, w2t, src, dst):
    NCH = E // SEG_CH
    GR = SEG_CH // 16

    @functools.partial(
        pl.kernel,
        out_type=jax.ShapeDtypeStruct((U, NP), f32),
        mesh=_mesh(),
        compiler_params=_sc_params(),
        scratch_types=[
            pltpu.VMEM((2, NP), f32),
            pltpu.VMEM((2, NP), f32),
            pltpu.VMEM((2, SEG_CH), f32),
            pltpu.VMEM((2, SEG_CH), f32),
            pltpu.VMEM((SEG_CH,), i32),
            pltpu.VMEM((SEG_CH,), i32),
            pltpu.VMEM((SEG_CH,), i32),
            pltpu.VMEM((SEG_CH,), i32),
            pltpu.SemaphoreType.DMA,
            pltpu.SemaphoreType.DMA,
            pltpu.SemaphoreType.DMA,
        ],
    )
    def k(x2t_hbm, w2t_hbm, src_hbm, dst_hbm, agg_hbm,
          x2c, acc, w2a, w2b, sa, sb, da, db, semA, semB, semC):
        cid = lax.axis_index("c")
        sid = lax.axis_index("s")
        wid = sid * 2 + cid
        c0 = wid * 2

        pltpu.async_copy(x2t_hbm.at[pl.ds(c0, 2)], x2c, semC).wait()

        neg = jnp.full((16,), -jnp.inf, f32)

        @pl.loop(0, NP // 16)
        def _(i):
            acc[0, pl.ds(i * 16, 16)] = neg
            acc[1, pl.ds(i * 16, 16)] = neg

        def start(ch, w2buf, sbuf, dbuf, sem):
            e0 = ch * SEG_CH
            pltpu.async_copy(
                w2t_hbm.at[pl.ds(c0, 2), pl.ds(e0, SEG_CH)], w2buf, sem)
            pltpu.async_copy(src_hbm.at[pl.ds(e0, SEG_CH)], sbuf, sem)
            pltpu.async_copy(dst_hbm.at[pl.ds(e0, SEG_CH)], dbuf, sem)

        def wait(w2buf, sbuf, dbuf, sem):
            pltpu.make_async_copy(
                w2t_hbm.at[pl.ds(0, 2), pl.ds(0, SEG_CH)], w2buf, sem).wait()
            pltpu.make_async_copy(src_hbm.at[pl.ds(0, SEG_CH)], sbuf,
                                  sem).wait()
            pltpu.make_async_copy(dst_hbm.at[pl.ds(0, SEG_CH)], dbuf,
                                  sem).wait()

        z16 = jnp.zeros((16,), i32)
        o16 = jnp.ones((16,), i32)
        lane = lax.iota(i32, 16)

        def process(w2buf, sbuf, dbuf):
            def group(b):
                s = sbuf[pl.ds(b, 16)]
                d = dbuf[pl.ds(b, 16)]
                xv0 = plsc.load_gather(x2c, [z16, d])
                xv1 = plsc.load_gather(x2c, [o16, d])
                m0 = w2buf[0, pl.ds(b, 16)] * xv0
                m1 = w2buf[1, pl.ds(b, 16)] * xv1

                ss, perm = plsc.sort_key_val(s, lane)
                m0s = _vtake(m0, perm)
                m1s = _vtake(m1, perm)

                for dd in (1, 2, 4, 8):
                    idx = jnp.maximum(lane - dd, 0)
                    ks = _vtake(ss, idx)
                    take = jnp.logical_and(ks == ss, lane >= dd)
                    p0 = _vtake(m0s, idx)
                    p1 = _vtake(m1s, idx)
                    m0s = jnp.where(take, jnp.maximum(m0s, p0), m0s)
                    m1s = jnp.where(take, jnp.maximum(m1s, p1), m1s)

                nxt = _vtake(ss, jnp.minimum(lane + 1, 15))
                last = jnp.logical_or(lane == 15, ss != nxt)

                a0 = plsc.load_gather(acc, [z16, ss])
                plsc.store_scatter(acc, [z16, ss],
                                   jnp.maximum(a0, m0s), mask=last)
                a1 = plsc.load_gather(acc, [o16, ss])
                plsc.store_scatter(acc, [o16, ss],
                                   jnp.maximum(a1, m1s), mask=last)

            @pl.loop(0, GR)
            def _(g):
                group(g * 16)

        start(0, w2a, sa, da, semA)
        start(1, w2b, sb, db, semB)

        @pl.loop(0, NCH, step=2)
        def _(ch):
            wait(w2a, sa, da, semA)
            process(w2a, sa, da)

            @pl.when(ch + 2 < NCH)
            def _():
                start(ch + 2, w2a, sa, da, semA)

            wait(w2b, sb, db, semB)
            process(w2b, sb, db)

            @pl.when(ch + 3 < NCH)
            def _():
                start(ch + 3, w2b, sb, db, semB)

        ninf = jnp.full((16,), -jnp.inf, f32)

        @pl.loop(0, NP // 16)
        def _(i):
            sl0 = (0, pl.ds(i * 16, 16))
            sl1 = (1, pl.ds(i * 16, 16))
            v0 = acc[sl0]
            v1 = acc[sl1]
            acc[sl0] = jnp.where(v0 == ninf, 0.0, v0)
            acc[sl1] = jnp.where(v1 == ninf, 0.0, v1)

        pltpu.async_copy(acc, agg_hbm.at[pl.ds(c0, 2)], semC).wait()

    return k(x2t, w2t, src, dst)


def kernel(x, edge_index, edge_attr, params):
    src = edge_index[0].astype(i32)
    dst = edge_index[1].astype(i32)
    src2d = src.reshape(1, E)
    dst2d = dst.reshape(1, E)

    w0v, b0v = params['v_lin0']
    w0e, b0e = params['e_lin0']

    xp = jnp.pad(x, ((0, NP - N), (0, 0)))
    xc = _pre_x(xp, w0v, b0v.reshape(1, U))
    wc = _pre_w(edge_attr, w0e.T, b0e.reshape(1, U))

    for i in range(DEPTH):
        w1v, b1v = params['v_lins1'][i]
        w2v, b2v = params['v_lins2'][i]
        w3v, b3v = params['v_lins3'][i]
        w4v, b4v = params['v_lins4'][i]
        wev, bev = params['e_lins0'][i]
        gn, bn_ = params['v_bns'][i]
        ge, be_ = params['e_bns'][i]

        x1, x3, x4, x2t = _node_mm(
            xc, w1v, b1v.reshape(1, U), w2v, b2v.reshape(U, 1),
            w3v, b3v.reshape(1, U), w4v, b4v.reshape(1, U))
        w1, w2t = _edge_mm(wc, wev, bev.reshape(1, U))
        g3, g4 = _gather_add(x3, x4, src2d, dst2d)
        aggt = _segmax(x2t, w2t, src, dst)
        te, ste = _edge_stats(w1, g3, g4)
        tn, stn = _node_stats(x1, aggt)
        xc = _finalize(xc, tn, stn, gn.reshape(1, U), bn_.reshape(1, U),
                       NP, NB)
        wc = _finalize(wc, te, ste, ge.reshape(1, U), be_.reshape(1, U),
                       E, EB)

    return xc[:N], wc


# speedup vs baseline: 2.1318x; 1.1145x over previous
import dataclasses
import functools

import jax
import jax.numpy as jnp
from jax import lax
from jax.experimental import pallas as pl
from jax.experimental.pallas import tpu as pltpu
from jax.experimental.pallas import tpu_sc as plsc

N = 10000
NP = 10240
E = 320000
U = 64
F = 128
DEPTH = 3
NB = 2048
EB = 2560
W_GATH = 128
SEG_CH = 1280
EPS = 1e-5

f32 = jnp.float32
i32 = jnp.int32


_GDN = lax.GatherDimensionNumbers(offset_dims=(), collapsed_slice_dims=(0,),
                                  start_index_map=(0,))


def _vtake(v, idx):
    return lax.gather(v, idx[:, None], _GDN, (1,),
                      mode=lax.GatherScatterMode.PROMISE_IN_BOUNDS)


def _eye(n):
    r = lax.broadcasted_iota(i32, (n, n), 0)
    c = lax.broadcasted_iota(i32, (n, n), 1)
    return jnp.where(r == c, 1.0, 0.0).astype(f32)


def _silu(v):
    return v * jax.nn.sigmoid(v)


def _dotT(a, b):
    return lax.dot_general(a, b, (((1,), (1,)), ((), ())),
                           preferred_element_type=f32)


def _pre_x_body(x_ref, w_ref, b_ref, o_ref):
    pid = pl.program_id(0)
    xb = x_ref[...]
    rows = pid * NB + lax.broadcasted_iota(i32, (NB, U), 0)
    v = _dotT(xb, w_ref[...]) + b_ref[...]
    o_ref[...] = jnp.where(rows < N, _silu(v), 0.0)


def _pre_x(xp, w0, b0r):
    return pl.pallas_call(
        _pre_x_body,
        grid=(NP // NB,),
        in_specs=[
            pl.BlockSpec((NB, F), lambda i: (i, 0)),
            pl.BlockSpec((U, F), lambda i: (0, 0)),
            pl.BlockSpec((1, U), lambda i: (0, 0)),
        ],
        out_specs=pl.BlockSpec((NB, U), lambda i: (i, 0)),
        out_shape=jax.ShapeDtypeStruct((NP, U), f32),
    )(xp, w0, b0r)


def _pre_w_body(ea_ref, wt_ref, b_ref, o_ref):
    ea = ea_ref[...]
    wt = wt_ref[...]
    v = ea[:, 0:1] * wt[0:1, :] + ea[:, 1:2] * wt[1:2, :] + b_ref[...]
    o_ref[...] = _silu(v)


def _pre_w(ea, wet, ber):
    return pl.pallas_call(
        _pre_w_body,
        grid=(E // EB,),
        in_specs=[
            pl.BlockSpec((EB, 2), lambda i: (i, 0)),
            pl.BlockSpec((2, U), lambda i: (0, 0)),
            pl.BlockSpec((1, U), lambda i: (0, 0)),
        ],
        out_specs=pl.BlockSpec((EB, U), lambda i: (i, 0)),
        out_shape=jax.ShapeDtypeStruct((E, U), f32),
    )(ea, wet, ber)


def _node_mm_body(x_ref, w1, b1, w2, b2t, w3, b3, w4, b4,
                  x1_ref, x3_ref, x4_ref, x2t_ref):
    pid = pl.program_id(0)
    xb = x_ref[...]
    rows = pid * NB + lax.broadcasted_iota(i32, (NB, U), 0)
    rmask = rows < N

    def lin(wr, br):
        return jnp.where(rmask, _dotT(xb, wr[...]) + br[...], 0.0)

    x1_ref[...] = lin(w1, b1)
    x3_ref[...] = lin(w3, b3)
    x4_ref[...] = lin(w4, b4)
    cols = pid * NB + lax.broadcasted_iota(i32, (U, NB), 1)
    x2t = _dotT(w2[...], xb) + b2t[...]
    x2t_ref[...] = jnp.where(cols < N, x2t, 0.0)


def _node_mm(x0, w1, b1, w2, b2t, w3, b3, w4, b4):
    full = lambda i: (0, 0)
    return pl.pallas_call(
        _node_mm_body,
        grid=(NP // NB,),
        in_specs=[
            pl.BlockSpec((NB, U), lambda i: (i, 0)),
            pl.BlockSpec((U, U), full), pl.BlockSpec((1, U), full),
            pl.BlockSpec((U, U), full), pl.BlockSpec((U, 1), full),
            pl.BlockSpec((U, U), full), pl.BlockSpec((1, U), full),
            pl.BlockSpec((U, U), full), pl.BlockSpec((1, U), full),
        ],
        out_specs=[
            pl.BlockSpec((NB, U), lambda i: (i, 0)),
            pl.BlockSpec((NB, U), lambda i: (i, 0)),
            pl.BlockSpec((NB, U), lambda i: (i, 0)),
            pl.BlockSpec((U, NB), lambda i: (0, i)),
        ],
        out_shape=[
            jax.ShapeDtypeStruct((NP, U), f32),
            jax.ShapeDtypeStruct((NP, U), f32),
            jax.ShapeDtypeStruct((NP, U), f32),
            jax.ShapeDtypeStruct((U, NP), f32),
        ],
    )(x0, w1, b1, w2, b2t, w3, b3, w4, b4)


def _edge_mm_body(w_ref, we, be, w1_ref, w2t_ref):
    wb = w_ref[...]
    w1_ref[...] = _dotT(wb, we[...]) + be[...]
    wbt = _dotT(_eye(U), wb)
    w2t_ref[...] = jax.nn.sigmoid(wbt)


def _edge_mm(w0, we, ber):
    full = lambda i: (0, 0)
    return pl.pallas_call(
        _edge_mm_body,
        grid=(E // EB,),
        in_specs=[
            pl.BlockSpec((EB, U), lambda i: (i, 0)),
            pl.BlockSpec((U, U), full), pl.BlockSpec((1, U), full),
        ],
        out_specs=[
            pl.BlockSpec((EB, U), lambda i: (i, 0)),
            pl.BlockSpec((U, EB), lambda i: (0, i)),
        ],
        out_shape=[
            jax.ShapeDtypeStruct((E, U), f32),
            jax.ShapeDtypeStruct((U, E), f32),
        ],
    )(w0, we, ber)


def _stats_body(nrows, a_ref, b_ref, c_ref, o_ref, st_ref):
    t = a_ref[...] + b_ref[...] + c_ref[...]
    o_ref[...] = t
    s = jnp.sum(t, axis=0, keepdims=True)
    q = jnp.sum(t * t, axis=0, keepdims=True)
    blk = jnp.concatenate([s, q, jnp.zeros((6, U), f32)], axis=0)

    @pl.when(pl.program_id(0) == 0)
    def _():
        st_ref[...] = blk

    @pl.when(pl.program_id(0) != 0)
    def _():
        st_ref[...] = st_ref[...] + blk


def _edge_stats(w1, g3, g4):
    return pl.pallas_call(
        functools.partial(_stats_body, E),
        grid=(E // EB,),
        in_specs=[
            pl.BlockSpec((EB, U), lambda i: (i, 0)),
            pl.BlockSpec((EB, U), lambda i: (i, 0)),
            pl.BlockSpec((EB, U), lambda i: (i, 0)),
        ],
        out_specs=[
            pl.BlockSpec((EB, U), lambda i: (i, 0)),
            pl.BlockSpec((8, U), lambda i: (0, 0)),
        ],
        out_shape=[
            jax.ShapeDtypeStruct((E, U), f32),
            jax.ShapeDtypeStruct((8, U), f32),
        ],
    )(w1, g3, g4)


def _node_stats_body(x1_ref, aggt_ref, o_ref, st_ref):
    p = aggt_ref[...]
    a = jnp.maximum(p[:U], p[U:])
    a = jnp.where(a == -jnp.inf, 0.0, a)
    t = x1_ref[...] + lax.dot_general(a, _eye(U),
                                      (((0,), (0,)), ((), ())),
                                      preferred_element_type=f32)
    o_ref[...] = t
    s = jnp.sum(t, axis=0, keepdims=True)
    q = jnp.sum(t * t, axis=0, keepdims=True)
    blk = jnp.concatenate([s, q, jnp.zeros((6, U), f32)], axis=0)

    @pl.when(pl.program_id(0) == 0)
    def _():
        st_ref[...] = blk

    @pl.when(pl.program_id(0) != 0)
    def _():
        st_ref[...] = st_ref[...] + blk


def _node_stats(x1, aggt):
    return pl.pallas_call(
        _node_stats_body,
        grid=(NP // NB,),
        in_specs=[
            pl.BlockSpec((NB, U), lambda i: (i, 0)),
            pl.BlockSpec((2 * U, NB), lambda i: (0, i)),
        ],
        out_specs=[
            pl.BlockSpec((NB, U), lambda i: (i, 0)),
            pl.BlockSpec((8, U), lambda i: (0, 0)),
        ],
        out_shape=[
            jax.ShapeDtypeStruct((NP, U), f32),
            jax.ShapeDtypeStruct((8, U), f32),
        ],
    )(x1, aggt)


def _fin_body(count, x0_ref, t_ref, st_ref, g_ref, b_ref, o_ref):
    st = st_ref[...]
    mean = st[0:1, :] / count
    var = st[1:2, :] / count - mean * mean
    istd = lax.rsqrt(var + EPS)
    t = t_ref[...]
    bn = (t - mean) * istd * g_ref[...] + b_ref[...]
    o_ref[...] = x0_ref[...] + _silu(bn)


def _finalize(x0, t, st, gam, bet, rows, blk):
    return pl.pallas_call(
        functools.partial(_fin_body, float(E if rows == E else N)),
        grid=(rows // blk,),
        in_specs=[
            pl.BlockSpec((blk, U), lambda i: (i, 0)),
            pl.BlockSpec((blk, U), lambda i: (i, 0)),
            pl.BlockSpec((8, U), lambda i: (0, 0)),
            pl.BlockSpec((1, U), lambda i: (0, 0)),
            pl.BlockSpec((1, U), lambda i: (0, 0)),
        ],
        out_specs=pl.BlockSpec((blk, U), lambda i: (i, 0)),
        out_shape=jax.ShapeDtypeStruct((rows, U), f32),
    )(x0, t, st, gam, bet)


_MESH = None


def _mesh():
    global _MESH
    if _MESH is None:
        _MESH = plsc.VectorSubcoreMesh(core_axis_name="c",
                                       subcore_axis_name="s")
    return _MESH


def _gather_add(x3, x4, src2d, dst2d):
    @functools.partial(
        pl.kernel,
        out_type=[jax.ShapeDtypeStruct((E, U), f32),
                  jax.ShapeDtypeStruct((E, U), f32)],
        mesh=_mesh(),
        compiler_params=_sc_params(tc_tiling=False),
        scratch_types=[pltpu.SemaphoreType.DMA, pltpu.SemaphoreType.DMA],
    )
    def k(x3_hbm, x4_hbm, src_hbm, dst_hbm, g3_hbm, g4_hbm, sem3, sem4):
        def body(s_v, d_v, o3_v, o4_v):
            c3 = pltpu.async_copy(x3_hbm.at[s_v.at[0]], o3_v, sem3)
            c4 = pltpu.async_copy(x4_hbm.at[d_v.at[0]], o4_v, sem4)
            c3.wait()
            c4.wait()

        pltpu.emit_pipeline(
            body,
            grid=(E // W_GATH,),
            in_specs=[
                pl.BlockSpec((1, W_GATH), lambda i: (0, i)),
                pl.BlockSpec((1, W_GATH), lambda i: (0, i)),
            ],
            out_specs=[pl.BlockSpec((W_GATH, U), lambda i: (i, 0)),
                       pl.BlockSpec((W_GATH, U), lambda i: (i, 0))],
            core_axis_name=("c", "s"),
            dimension_semantics=(pltpu.PARALLEL,),
        )(src_hbm, dst_hbm, g3_hbm, g4_hbm)

    return k(x3, x4, src2d, dst2d)


def _sc_params(tc_tiling=True):
    cp = pltpu.CompilerParams()
    if "needs_layout_passes" in pltpu.CompilerParams.__dataclass_fields__:
        cp = dataclasses.replace(cp, needs_layout_passes=False)
    if not tc_tiling:
        cp = dataclasses.replace(cp, use_tc_tiling_on_sc=False)
    return cp


def _segmax(x2t, w2t, src, dst):
    SH = E // 2
    NCH = SH // SEG_CH
    GR = SEG_CH // 16

    @functools.partial(
        pl.kernel,
        out_type=jax.ShapeDtypeStruct((2 * U, NP), f32),
        mesh=_mesh(),
        compiler_params=_sc_params(),
        scratch_types=[
            pltpu.VMEM((4, NP), f32),
            pltpu.VMEM((4, NP), f32),
            pltpu.VMEM((4, SEG_CH), f32),
            pltpu.VMEM((4, SEG_CH), f32),
            pltpu.VMEM((SEG_CH,), i32),
            pltpu.VMEM((SEG_CH,), i32),
            pltpu.VMEM((SEG_CH,), i32),
            pltpu.VMEM((SEG_CH,), i32),
            pltpu.SemaphoreType.DMA,
            pltpu.SemaphoreType.DMA,
            pltpu.SemaphoreType.DMA,
        ],
    )
    def k(x2t_hbm, w2t_hbm, src_hbm, dst_hbm, agg_hbm,
          x2c, acc, w2a, w2b, sa, sb, da, db, semA, semB, semC):
        cid = lax.axis_index("c")
        sid = lax.axis_index("s")
        wid = sid * 2 + cid
        shard = wid & 1
        c0 = (wid // 2) * 4
        eb0 = shard * SH

        pltpu.async_copy(x2t_hbm.at[pl.ds(c0, 4)], x2c, semC).wait()

        neg = jnp.full((16,), -jnp.inf, f32)

        @pl.loop(0, NP // 16)
        def _(i):
            for r in range(4):
                acc[r, pl.ds(i * 16, 16)] = neg

        def start(ch, w2buf, sbuf, dbuf, sem):
            e0 = eb0 + ch * SEG_CH
            pltpu.async_copy(
                w2t_hbm.at[pl.ds(c0, 4), pl.ds(e0, SEG_CH)], w2buf, sem)
            pltpu.async_copy(src_hbm.at[pl.ds(e0, SEG_CH)], sbuf, sem)
            pltpu.async_copy(dst_hbm.at[pl.ds(e0, SEG_CH)], dbuf, sem)

        def wait(w2buf, sbuf, dbuf, sem):
            pltpu.make_async_copy(
                w2t_hbm.at[pl.ds(0, 4), pl.ds(0, SEG_CH)], w2buf, sem).wait()
            pltpu.make_async_copy(src_hbm.at[pl.ds(0, SEG_CH)], sbuf,
                                  sem).wait()
            pltpu.make_async_copy(dst_hbm.at[pl.ds(0, SEG_CH)], dbuf,
                                  sem).wait()

        lane = lax.iota(i32, 16)
        ridx = [jnp.full((16,), r, i32) for r in range(4)]

        def process(w2buf, sbuf, dbuf):
            def group(b):
                s = sbuf[pl.ds(b, 16)]
                d = dbuf[pl.ds(b, 16)]
                mv = [w2buf[r, pl.ds(b, 16)] *
                      plsc.load_gather(x2c, [ridx[r], d]) for r in range(4)]

                ss, perm = plsc.sort_key_val(s, lane)
                mv = [_vtake(m, perm) for m in mv]

                for dd in (1, 2, 4, 8):
                    idx = jnp.maximum(lane - dd, 0)
                    ks = _vtake(ss, idx)
                    take = jnp.logical_and(ks == ss, lane >= dd)
                    mv = [jnp.where(take, jnp.maximum(m, _vtake(m, idx)), m)
                          for m in mv]

                nxt = _vtake(ss, jnp.minimum(lane + 1, 15))
                last = jnp.logical_or(lane == 15, ss != nxt)

                for r in range(4):
                    a = plsc.load_gather(acc, [ridx[r], ss])
                    plsc.store_scatter(acc, [ridx[r], ss],
                                       jnp.maximum(a, mv[r]), mask=last)

            @pl.loop(0, GR)
            def _(g):
                group(g * 16)

        start(0, w2a, sa, da, semA)
        start(1, w2b, sb, db, semB)

        @pl.loop(0, NCH - 1, step=2)
        def _(ch):
            wait(w2a, sa, da, semA)
            process(w2a, sa, da)

            @pl.when(ch + 2 < NCH)
            def _():
                start(ch + 2, w2a, sa, da, semA)

            wait(w2b, sb, db, semB)
            process(w2b, sb, db)

            @pl.when(ch + 3 < NCH)
            def _():
                start(ch + 3, w2b, sb, db, semB)

        if NCH % 2 == 1:
            wait(w2a, sa, da, semA)
            process(w2a, sa, da)

        pltpu.async_copy(acc, agg_hbm.at[pl.ds(shard * U + c0, 4)],
                         semC).wait()

    return k(x2t, w2t, src, dst)


def kernel(x, edge_index, edge_attr, params):
    src = edge_index[0].astype(i32)
    dst = edge_index[1].astype(i32)
    src2d = src.reshape(1, E)
    dst2d = dst.reshape(1, E)

    w0v, b0v = params['v_lin0']
    w0e, b0e = params['e_lin0']

    xp = jnp.pad(x, ((0, NP - N), (0, 0)))
    xc = _pre_x(xp, w0v, b0v.reshape(1, U))
    wc = _pre_w(edge_attr, w0e.T, b0e.reshape(1, U))

    for i in range(DEPTH):
        w1v, b1v = params['v_lins1'][i]
        w2v, b2v = params['v_lins2'][i]
        w3v, b3v = params['v_lins3'][i]
        w4v, b4v = params['v_lins4'][i]
        wev, bev = params['e_lins0'][i]
        gn, bn_ = params['v_bns'][i]
        ge, be_ = params['e_bns'][i]

        x1, x3, x4, x2t = _node_mm(
            xc, w1v, b1v.reshape(1, U), w2v, b2v.reshape(U, 1),
            w3v, b3v.reshape(1, U), w4v, b4v.reshape(1, U))
        w1, w2t = _edge_mm(wc, wev, bev.reshape(1, U))
        g3, g4 = _gather_add(x3, x4, src2d, dst2d)
        aggt = _segmax(x2t, w2t, src, dst)
        te, ste = _edge_stats(w1, g3, g4)
        tn, stn = _node_stats(x1, aggt)
        xc = _finalize(xc, tn, stn, gn.reshape(1, U), bn_.reshape(1, U),
                       NP, NB)
        wc = _finalize(wc, te, ste, ge.reshape(1, U), be_.reshape(1, U),
                       E, EB)

    return xc[:N], wc


# speedup vs baseline: 2.9190x; 1.3692x over previous
import dataclasses
import functools

import jax
import jax.numpy as jnp
from jax import lax
from jax.experimental import pallas as pl
from jax.experimental.pallas import tpu as pltpu
from jax.experimental.pallas import tpu_sc as plsc

N = 10000
NP = 10240
E = 320000
E2 = E // 2
EB2 = 1280
U = 64
F = 128
DEPTH = 3
NB = 2048
EB = 2560
W_GATH = 128
SEG_CH = 1280
EPS = 1e-5

f32 = jnp.float32
i32 = jnp.int32


_GDN = lax.GatherDimensionNumbers(offset_dims=(), collapsed_slice_dims=(0,),
                                  start_index_map=(0,))


def _vtake(v, idx):
    return lax.gather(v, idx[:, None], _GDN, (1,),
                      mode=lax.GatherScatterMode.PROMISE_IN_BOUNDS)


def _eye(n):
    r = lax.broadcasted_iota(i32, (n, n), 0)
    c = lax.broadcasted_iota(i32, (n, n), 1)
    return jnp.where(r == c, 1.0, 0.0).astype(f32)


def _silu(v):
    return v * jax.nn.sigmoid(v)


def _dotT(a, b):
    return lax.dot_general(a, b, (((1,), (1,)), ((), ())),
                           preferred_element_type=f32)


def _pre_x_body(x_ref, w_ref, b_ref, o_ref):
    pid = pl.program_id(0)
    xb = x_ref[...]
    rows = pid * NB + lax.broadcasted_iota(i32, (NB, U), 0)
    v = _dotT(xb, w_ref[...]) + b_ref[...]
    o_ref[...] = jnp.where(rows < N, _silu(v), 0.0)


def _pre_x(xp, w0, b0r):
    return pl.pallas_call(
        _pre_x_body,
        grid=(NP // NB,),
        in_specs=[
            pl.BlockSpec((NB, F), lambda i: (i, 0)),
            pl.BlockSpec((U, F), lambda i: (0, 0)),
            pl.BlockSpec((1, U), lambda i: (0, 0)),
        ],
        out_specs=pl.BlockSpec((NB, U), lambda i: (i, 0)),
        out_shape=jax.ShapeDtypeStruct((NP, U), f32),
    )(xp, w0, b0r)


def _pre_w_body(ea_ref, wt_ref, b_ref, o_ref):
    ea = ea_ref[...]
    wt = wt_ref[...]
    b = b_ref[...]
    ve = ea[:, 0:1] * wt[0:1, :] + ea[:, 1:2] * wt[1:2, :] + b
    vo = ea[:, 2:3] * wt[0:1, :] + ea[:, 3:4] * wt[1:2, :] + b
    o_ref[...] = _silu(jnp.concatenate([ve, vo], axis=1))


def _pre_w(ea4, wet, ber):
    return pl.pallas_call(
        _pre_w_body,
        grid=(E2 // EB2,),
        in_specs=[
            pl.BlockSpec((EB2, 4), lambda i: (i, 0)),
            pl.BlockSpec((2, U), lambda i: (0, 0)),
            pl.BlockSpec((1, U), lambda i: (0, 0)),
        ],
        out_specs=pl.BlockSpec((EB2, 2 * U), lambda i: (i, 0)),
        out_shape=jax.ShapeDtypeStruct((E2, 2 * U), f32),
    )(ea4, wet, ber)


def _node_mm_body(x_ref, w1, b1, w2, b2t, w3, b3, w4, b4,
                  x1_ref, x3_ref, x4_ref, x2t_ref):
    pid = pl.program_id(0)
    xb = x_ref[...]
    rows = pid * NB + lax.broadcasted_iota(i32, (NB, U), 0)
    rmask = rows < N

    def lin(wr, br):
        return jnp.where(rmask, _dotT(xb, wr[...]) + br[...], 0.0)

    x1_ref[...] = lin(w1, b1)
    x3_ref[...] = lin(w3, b3)
    x4_ref[...] = lin(w4, b4)
    cols = pid * NB + lax.broadcasted_iota(i32, (U, NB), 1)
    x2t = _dotT(w2[...], xb) + b2t[...]
    x2t_ref[...] = jnp.where(cols < N, x2t, 0.0)


def _node_mm(x0, w1, b1, w2, b2t, w3, b3, w4, b4):
    full = lambda i: (0, 0)
    return pl.pallas_call(
        _node_mm_body,
        grid=(NP // NB,),
        in_specs=[
            pl.BlockSpec((NB, U), lambda i: (i, 0)),
            pl.BlockSpec((U, U), full), pl.BlockSpec((1, U), full),
            pl.BlockSpec((U, U), full), pl.BlockSpec((U, 1), full),
            pl.BlockSpec((U, U), full), pl.BlockSpec((1, U), full),
            pl.BlockSpec((U, U), full), pl.BlockSpec((1, U), full),
        ],
        out_specs=[
            pl.BlockSpec((NB, U), lambda i: (i, 0)),
            pl.BlockSpec((NB, U), lambda i: (i, 0)),
            pl.BlockSpec((NB, U), lambda i: (i, 0)),
            pl.BlockSpec((U, NB), lambda i: (0, i)),
        ],
        out_shape=[
            jax.ShapeDtypeStruct((NP, U), f32),
            jax.ShapeDtypeStruct((NP, U), f32),
            jax.ShapeDtypeStruct((NP, U), f32),
            jax.ShapeDtypeStruct((U, NP), f32),
        ],
    )(x0, w1, b1, w2, b2t, w3, b3, w4, b4)


def _edge_mm_body(w_ref, wbd, be, w1_ref, w2t_ref):
    wb = w_ref[...]
    w1_ref[...] = jnp.dot(wb, wbd[...],
                          preferred_element_type=f32) + be[...]
    wbt = _dotT(_eye(2 * U), wb)
    w2t_ref[...] = jax.nn.sigmoid(wbt)


def _edge_mm(w0, wbd, ber):
    full = lambda i: (0, 0)
    return pl.pallas_call(
        _edge_mm_body,
        grid=(E2 // EB2,),
        in_specs=[
            pl.BlockSpec((EB2, 2 * U), lambda i: (i, 0)),
            pl.BlockSpec((2 * U, 2 * U), full),
            pl.BlockSpec((1, 2 * U), full),
        ],
        out_specs=[
            pl.BlockSpec((EB2, 2 * U), lambda i: (i, 0)),
            pl.BlockSpec((2 * U, EB2), lambda i: (0, i)),
        ],
        out_shape=[
            jax.ShapeDtypeStruct((E2, 2 * U), f32),
            jax.ShapeDtypeStruct((2 * U, E2), f32),
        ],
    )(w0, wbd, ber)


def _stats_body(a_ref, b_ref, c_ref, o_ref, st_ref):
    t = a_ref[...] + b_ref[...] + c_ref[...]
    o_ref[...] = t
    s = jnp.sum(t, axis=0, keepdims=True)
    q = jnp.sum(t * t, axis=0, keepdims=True)
    blk = jnp.concatenate([s, q, jnp.zeros((6, 2 * U), f32)], axis=0)

    @pl.when(pl.program_id(0) == 0)
    def _():
        st_ref[...] = blk

    @pl.when(pl.program_id(0) != 0)
    def _():
        st_ref[...] = st_ref[...] + blk


def _edge_stats(w1, g3, g4):
    return pl.pallas_call(
        _stats_body,
        grid=(E2 // EB2,),
        in_specs=[
            pl.BlockSpec((EB2, 2 * U), lambda i: (i, 0)),
            pl.BlockSpec((EB2, 2 * U), lambda i: (i, 0)),
            pl.BlockSpec((EB2, 2 * U), lambda i: (i, 0)),
        ],
        out_specs=[
            pl.BlockSpec((EB2, 2 * U), lambda i: (i, 0)),
            pl.BlockSpec((8, 2 * U), lambda i: (0, 0)),
        ],
        out_shape=[
            jax.ShapeDtypeStruct((E2, 2 * U), f32),
            jax.ShapeDtypeStruct((8, 2 * U), f32),
        ],
    )(w1, g3, g4)


def _node_stats_body(x1_ref, aggt_ref, o_ref, st_ref):
    p = aggt_ref[...]
    a = jnp.maximum(p[:U], p[U:])
    a = jnp.where(a == -jnp.inf, 0.0, a)
    t = x1_ref[...] + lax.dot_general(a, _eye(U),
                                      (((0,), (0,)), ((), ())),
                                      preferred_element_type=f32)
    o_ref[...] = t
    s = jnp.sum(t, axis=0, keepdims=True)
    q = jnp.sum(t * t, axis=0, keepdims=True)
    blk = jnp.concatenate([s, q, jnp.zeros((6, U), f32)], axis=0)

    @pl.when(pl.program_id(0) == 0)
    def _():
        st_ref[...] = blk

    @pl.when(pl.program_id(0) != 0)
    def _():
        st_ref[...] = st_ref[...] + blk


def _node_stats(x1, aggt):
    return pl.pallas_call(
        _node_stats_body,
        grid=(NP // NB,),
        in_specs=[
            pl.BlockSpec((NB, U), lambda i: (i, 0)),
            pl.BlockSpec((2 * U, NB), lambda i: (0, i)),
        ],
        out_specs=[
            pl.BlockSpec((NB, U), lambda i: (i, 0)),
            pl.BlockSpec((8, U), lambda i: (0, 0)),
        ],
        out_shape=[
            jax.ShapeDtypeStruct((NP, U), f32),
            jax.ShapeDtypeStruct((8, U), f32),
        ],
    )(x1, aggt)


def _fin_edge_body(x0_ref, t_ref, st_ref, g_ref, b_ref, o_ref):
    st = st_ref[...]
    s64 = st[0:1, :U] + st[0:1, U:]
    q64 = st[1:2, :U] + st[1:2, U:]
    mean = s64 / E
    var = q64 / E - mean * mean
    istd = lax.rsqrt(var + EPS)
    mean2 = jnp.concatenate([mean, mean], axis=1)
    istd2 = jnp.concatenate([istd, istd], axis=1)
    t = t_ref[...]
    bn = (t - mean2) * istd2 * g_ref[...] + b_ref[...]
    o_ref[...] = x0_ref[...] + _silu(bn)


def _fin_edge(w0, t, st, gam2, bet2):
    return pl.pallas_call(
        _fin_edge_body,
        grid=(E2 // EB2,),
        in_specs=[
            pl.BlockSpec((EB2, 2 * U), lambda i: (i, 0)),
            pl.BlockSpec((EB2, 2 * U), lambda i: (i, 0)),
            pl.BlockSpec((8, 2 * U), lambda i: (0, 0)),
            pl.BlockSpec((1, 2 * U), lambda i: (0, 0)),
            pl.BlockSpec((1, 2 * U), lambda i: (0, 0)),
        ],
        out_specs=pl.BlockSpec((EB2, 2 * U), lambda i: (i, 0)),
        out_shape=jax.ShapeDtypeStruct((E2, 2 * U), f32),
    )(w0, t, st, gam2, bet2)


def _fin_body(count, x0_ref, t_ref, st_ref, g_ref, b_ref, o_ref):
    st = st_ref[...]
    mean = st[0:1, :] / count
    var = st[1:2, :] / count - mean * mean
    istd = lax.rsqrt(var + EPS)
    t = t_ref[...]
    bn = (t - mean) * istd * g_ref[...] + b_ref[...]
    o_ref[...] = x0_ref[...] + _silu(bn)


def _finalize(x0, t, st, gam, bet, rows, blk):
    return pl.pallas_call(
        functools.partial(_fin_body, float(E if rows == E else N)),
        grid=(rows // blk,),
        in_specs=[
            pl.BlockSpec((blk, U), lambda i: (i, 0)),
            pl.BlockSpec((blk, U), lambda i: (i, 0)),
            pl.BlockSpec((8, U), lambda i: (0, 0)),
            pl.BlockSpec((1, U), lambda i: (0, 0)),
            pl.BlockSpec((1, U), lambda i: (0, 0)),
        ],
        out_specs=pl.BlockSpec((blk, U), lambda i: (i, 0)),
        out_shape=jax.ShapeDtypeStruct((rows, U), f32),
    )(x0, t, st, gam, bet)


_MESH = None


def _mesh():
    global _MESH
    if _MESH is None:
        _MESH = plsc.VectorSubcoreMesh(core_axis_name="c",
                                       subcore_axis_name="s")
    return _MESH


def _gather_add(x3, x4, src2d, dst2d):
    @functools.partial(
        pl.kernel,
        out_type=[jax.ShapeDtypeStruct((E, U), f32),
                  jax.ShapeDtypeStruct((E, U), f32)],
        mesh=_mesh(),
        compiler_params=_sc_params(tc_tiling=False),
        scratch_types=[pltpu.SemaphoreType.DMA, pltpu.SemaphoreType.DMA],
    )
    def k(x3_hbm, x4_hbm, src_hbm, dst_hbm, g3_hbm, g4_hbm, sem3, sem4):
        def body(s_v, d_v, o3_v, o4_v):
            c3 = pltpu.async_copy(x3_hbm.at[s_v.at[0]], o3_v, sem3)
            c4 = pltpu.async_copy(x4_hbm.at[d_v.at[0]], o4_v, sem4)
            c3.wait()
            c4.wait()

        pltpu.emit_pipeline(
            body,
            grid=(E // W_GATH,),
            in_specs=[
                pl.BlockSpec((1, W_GATH), lambda i: (0, i)),
                pl.BlockSpec((1, W_GATH), lambda i: (0, i)),
            ],
            out_specs=[pl.BlockSpec((W_GATH, U), lambda i: (i, 0)),
                       pl.BlockSpec((W_GATH, U), lambda i: (i, 0))],
            core_axis_name=("c", "s"),
            dimension_semantics=(pltpu.PARALLEL,),
        )(src_hbm, dst_hbm, g3_hbm, g4_hbm)

    return k(x3, x4, src2d, dst2d)


def _sc_params(tc_tiling=True):
    cp = pltpu.CompilerParams()
    if "needs_layout_passes" in pltpu.CompilerParams.__dataclass_fields__:
        cp = dataclasses.replace(cp, needs_layout_passes=False)
    if not tc_tiling:
        cp = dataclasses.replace(cp, use_tc_tiling_on_sc=False)
    return cp


def _segmax(x2t, w2t2, src_eo, dst_eo):
    NCH = E2 // SEG_CH
    GR = SEG_CH // 16

    @functools.partial(
        pl.kernel,
        out_type=jax.ShapeDtypeStruct((2 * U, NP), f32),
        mesh=_mesh(),
        compiler_params=_sc_params(),
        scratch_types=[
            pltpu.VMEM((4, NP), f32),
            pltpu.VMEM((4, NP), f32),
            pltpu.VMEM((4, SEG_CH), f32),
            pltpu.VMEM((4, SEG_CH), f32),
            pltpu.VMEM((SEG_CH,), i32),
            pltpu.VMEM((SEG_CH,), i32),
            pltpu.VMEM((SEG_CH,), i32),
            pltpu.VMEM((SEG_CH,), i32),
            pltpu.SemaphoreType.DMA,
            pltpu.SemaphoreType.DMA,
            pltpu.SemaphoreType.DMA,
        ],
    )
    def k(x2t_hbm, w2t_hbm, src_hbm, dst_hbm, agg_hbm,
          x2c, acc, w2a, w2b, sa, sb, da, db, semA, semB, semC):
        cid = lax.axis_index("c")
        sid = lax.axis_index("s")
        wid = sid * 2 + cid
        shard = wid & 1
        c0 = (wid // 2) * 4
        w2row = shard * U + c0

        pltpu.async_copy(x2t_hbm.at[pl.ds(c0, 4)], x2c, semC).wait()

        neg = jnp.full((16,), -jnp.inf, f32)

        @pl.loop(0, NP // 16)
        def _(i):
            for r in range(4):
                acc[r, pl.ds(i * 16, 16)] = neg

        def start(ch, w2buf, sbuf, dbuf, sem):
            e0 = ch * SEG_CH
            pltpu.async_copy(
                w2t_hbm.at[pl.ds(w2row, 4), pl.ds(e0, SEG_CH)], w2buf, sem)
            pltpu.async_copy(src_hbm.at[shard, pl.ds(e0, SEG_CH)], sbuf,
                             sem)
            pltpu.async_copy(dst_hbm.at[shard, pl.ds(e0, SEG_CH)], dbuf,
                             sem)

        def wait(w2buf, sbuf, dbuf, sem):
            pltpu.make_async_copy(
                w2t_hbm.at[pl.ds(0, 4), pl.ds(0, SEG_CH)], w2buf, sem).wait()
            pltpu.make_async_copy(src_hbm.at[0, pl.ds(0, SEG_CH)], sbuf,
                                  sem).wait()
            pltpu.make_async_copy(dst_hbm.at[0, pl.ds(0, SEG_CH)], dbuf,
                                  sem).wait()

        lane = lax.iota(i32, 16)
        ridx = [jnp.full((16,), r, i32) for r in range(4)]

        def process(w2buf, sbuf, dbuf):
            def group(b):
                s = sbuf[pl.ds(b, 16)]
                d = dbuf[pl.ds(b, 16)]
                mv = [w2buf[r, pl.ds(b, 16)] *
                      plsc.load_gather(x2c, [ridx[r], d]) for r in range(4)]

                ss, perm = plsc.sort_key_val(s, lane)
                mv = [_vtake(m, perm) for m in mv]

                for dd in (1, 2, 4, 8):
                    idx = jnp.maximum(lane - dd, 0)
                    ks = _vtake(ss, idx)
                    take = jnp.logical_and(ks == ss, lane >= dd)
                    mv = [jnp.where(take, jnp.maximum(m, _vtake(m, idx)), m)
                          for m in mv]

                nxt = _vtake(ss, jnp.minimum(lane + 1, 15))
                last = jnp.logical_or(lane == 15, ss != nxt)

                for r in range(4):
                    a = plsc.load_gather(acc, [ridx[r], ss])
                    plsc.store_scatter(acc, [ridx[r], ss],
                                       jnp.maximum(a, mv[r]), mask=last)

            @pl.loop(0, GR)
            def _(g):
                group(g * 16)

        start(0, w2a, sa, da, semA)
        start(1, w2b, sb, db, semB)

        @pl.loop(0, NCH - 1, step=2)
        def _(ch):
            wait(w2a, sa, da, semA)
            process(w2a, sa, da)

            @pl.when(ch + 2 < NCH)
            def _():
                start(ch + 2, w2a, sa, da, semA)

            wait(w2b, sb, db, semB)
            process(w2b, sb, db)

            @pl.when(ch + 3 < NCH)
            def _():
                start(ch + 3, w2b, sb, db, semB)

        if NCH % 2 == 1:
            wait(w2a, sa, da, semA)
            process(w2a, sa, da)

        pltpu.async_copy(acc, agg_hbm.at[pl.ds(shard * U + c0, 4)],
                         semC).wait()

    return k(x2t, w2t2, src_eo, dst_eo)


def kernel(x, edge_index, edge_attr, params):
    src = edge_index[0].astype(i32)
    dst = edge_index[1].astype(i32)
    src2d = src.reshape(1, E)
    dst2d = dst.reshape(1, E)
    src_eo = src.reshape(E2, 2).T
    dst_eo = dst.reshape(E2, 2).T

    w0v, b0v = params['v_lin0']
    w0e, b0e = params['e_lin0']

    xp = jnp.pad(x, ((0, NP - N), (0, 0)))
    xc = _pre_x(xp, w0v, b0v.reshape(1, U))
    wc = _pre_w(edge_attr.reshape(E2, 4), w0e.T, b0e.reshape(1, U))

    for i in range(DEPTH):
        w1v, b1v = params['v_lins1'][i]
        w2v, b2v = params['v_lins2'][i]
        w3v, b3v = params['v_lins3'][i]
        w4v, b4v = params['v_lins4'][i]
        wev, bev = params['e_lins0'][i]
        gn, bn_ = params['v_bns'][i]
        ge, be_ = params['e_bns'][i]

        z = jnp.zeros((U, U), f32)
        wbd = jnp.block([[wev.T, z], [z, wev.T]])
        bev2 = jnp.tile(bev, 2).reshape(1, 2 * U)

        x1, x3, x4, x2t = _node_mm(
            xc, w1v, b1v.reshape(1, U), w2v, b2v.reshape(U, 1),
            w3v, b3v.reshape(1, U), w4v, b4v.reshape(1, U))
        w1, w2t2 = _edge_mm(wc, wbd, bev2)
        g3, g4 = _gather_add(x3, x4, src2d, dst2d)
        aggt = _segmax(x2t, w2t2, src_eo, dst_eo)
        te, ste = _edge_stats(w1, g3.reshape(E2, 2 * U),
                              g4.reshape(E2, 2 * U))
        tn, stn = _node_stats(x1, aggt)
        xc = _finalize(xc, tn, stn, gn.reshape(1, U), bn_.reshape(1, U),
                       NP, NB)
        wc = _fin_edge(wc, te, ste, jnp.tile(ge, 2).reshape(1, 2 * U),
                       jnp.tile(be_, 2).reshape(1, 2 * U))

    return xc[:N], wc.reshape(E, U)


# speedup vs baseline: 3.0870x; 1.0576x over previous
import dataclasses
import functools

import jax
import jax.numpy as jnp
from jax import lax
from jax.experimental import pallas as pl
from jax.experimental.pallas import tpu as pltpu
from jax.experimental.pallas import tpu_sc as plsc

N = 10000
NP = 10240
E = 320000
E2 = E // 2
EB2 = 1280
U = 64
F = 128
DEPTH = 3
NB = 2048
EB = 2560
W_GATH = 128
SEG_CH = 1280
EPS = 1e-5

f32 = jnp.float32
i32 = jnp.int32


_GDN = lax.GatherDimensionNumbers(offset_dims=(), collapsed_slice_dims=(0,),
                                  start_index_map=(0,))


def _vtake(v, idx):
    return lax.gather(v, idx[:, None], _GDN, (1,),
                      mode=lax.GatherScatterMode.PROMISE_IN_BOUNDS)


def _eye(n):
    r = lax.broadcasted_iota(i32, (n, n), 0)
    c = lax.broadcasted_iota(i32, (n, n), 1)
    return jnp.where(r == c, 1.0, 0.0).astype(f32)


def _silu(v):
    return v * jax.nn.sigmoid(v)


def _dotT(a, b):
    return lax.dot_general(a, b, (((1,), (1,)), ((), ())),
                           preferred_element_type=f32)


def _pre_x_body(x_ref, w_ref, b_ref, o_ref):
    pid = pl.program_id(0)
    xb = x_ref[...]
    rows = pid * NB + lax.broadcasted_iota(i32, (NB, U), 0)
    v = _dotT(xb, w_ref[...]) + b_ref[...]
    o_ref[...] = jnp.where(rows < N, _silu(v), 0.0)


def _pre_x(xp, w0, b0r):
    return pl.pallas_call(
        _pre_x_body,
        grid=(NP // NB,),
        in_specs=[
            pl.BlockSpec((NB, F), lambda i: (i, 0)),
            pl.BlockSpec((U, F), lambda i: (0, 0)),
            pl.BlockSpec((1, U), lambda i: (0, 0)),
        ],
        out_specs=pl.BlockSpec((NB, U), lambda i: (i, 0)),
        out_shape=jax.ShapeDtypeStruct((NP, U), f32),
    )(xp, w0, b0r)


def _pre_w_body(ea_ref, wt_ref, b_ref, o_ref):
    ea = ea_ref[...]
    wt = wt_ref[...]
    b = b_ref[...]
    ve = ea[:, 0:1] * wt[0:1, :] + ea[:, 1:2] * wt[1:2, :] + b
    vo = ea[:, 2:3] * wt[0:1, :] + ea[:, 3:4] * wt[1:2, :] + b
    o_ref[...] = _silu(jnp.concatenate([ve, vo], axis=1))


def _pre_w(ea4, wet, ber):
    return pl.pallas_call(
        _pre_w_body,
        grid=(E2 // EB2,),
        in_specs=[
            pl.BlockSpec((EB2, 4), lambda i: (i, 0)),
            pl.BlockSpec((2, U), lambda i: (0, 0)),
            pl.BlockSpec((1, U), lambda i: (0, 0)),
        ],
        out_specs=pl.BlockSpec((EB2, 2 * U), lambda i: (i, 0)),
        out_shape=jax.ShapeDtypeStruct((E2, 2 * U), f32),
    )(ea4, wet, ber)


def _node_mm_body(x_ref, w1, b1, w2, b2t, w3, b3, w4, b4,
                  x1_ref, x3_ref, x4_ref, x2t_ref):
    pid = pl.program_id(0)
    xb = x_ref[...]
    rows = pid * NB + lax.broadcasted_iota(i32, (NB, U), 0)
    rmask = rows < N

    def lin(wr, br):
        return jnp.where(rmask, _dotT(xb, wr[...]) + br[...], 0.0)

    x1_ref[...] = lin(w1, b1)
    x3_ref[...] = lin(w3, b3)
    x4_ref[...] = lin(w4, b4)
    cols = pid * NB + lax.broadcasted_iota(i32, (U, NB), 1)
    x2t = _dotT(w2[...], xb) + b2t[...]
    x2t_ref[...] = jnp.where(cols < N, x2t, 0.0)


def _node_mm(x0, w1, b1, w2, b2t, w3, b3, w4, b4):
    full = lambda i: (0, 0)
    return pl.pallas_call(
        _node_mm_body,
        grid=(NP // NB,),
        in_specs=[
            pl.BlockSpec((NB, U), lambda i: (i, 0)),
            pl.BlockSpec((U, U), full), pl.BlockSpec((1, U), full),
            pl.BlockSpec((U, U), full), pl.BlockSpec((U, 1), full),
            pl.BlockSpec((U, U), full), pl.BlockSpec((1, U), full),
            pl.BlockSpec((U, U), full), pl.BlockSpec((1, U), full),
        ],
        out_specs=[
            pl.BlockSpec((NB, U), lambda i: (i, 0)),
            pl.BlockSpec((NB, U), lambda i: (i, 0)),
            pl.BlockSpec((NB, U), lambda i: (i, 0)),
            pl.BlockSpec((U, NB), lambda i: (0, i)),
        ],
        out_shape=[
            jax.ShapeDtypeStruct((NP, U), f32),
            jax.ShapeDtypeStruct((NP, U), f32),
            jax.ShapeDtypeStruct((NP, U), f32),
            jax.ShapeDtypeStruct((U, NP), f32),
        ],
    )(x0, w1, b1, w2, b2t, w3, b3, w4, b4)


def _edge_mm_body(w_ref, wbd, be, w1_ref, w2t_ref):
    wb = w_ref[...]
    w1_ref[...] = jnp.dot(wb, wbd[...],
                          preferred_element_type=f32) + be[...]
    wbt = _dotT(_eye(2 * U), wb)
    w2t_ref[...] = jax.nn.sigmoid(wbt)


def _edge_mm(w0, wbd, ber):
    full = lambda i: (0, 0)
    return pl.pallas_call(
        _edge_mm_body,
        grid=(E2 // EB2,),
        in_specs=[
            pl.BlockSpec((EB2, 2 * U), lambda i: (i, 0)),
            pl.BlockSpec((2 * U, 2 * U), full),
            pl.BlockSpec((1, 2 * U), full),
        ],
        out_specs=[
            pl.BlockSpec((EB2, 2 * U), lambda i: (i, 0)),
            pl.BlockSpec((2 * U, EB2), lambda i: (0, i)),
        ],
        out_shape=[
            jax.ShapeDtypeStruct((E2, 2 * U), f32),
            jax.ShapeDtypeStruct((2 * U, E2), f32),
        ],
    )(w0, wbd, ber)


def _stats_body(a_ref, b_ref, c_ref, o_ref, st_ref):
    t = a_ref[...] + b_ref[...] + c_ref[...]
    o_ref[...] = t
    s = jnp.sum(t, axis=0, keepdims=True)
    q = jnp.sum(t * t, axis=0, keepdims=True)
    blk = jnp.concatenate([s, q, jnp.zeros((6, 2 * U), f32)], axis=0)

    @pl.when(pl.program_id(0) == 0)
    def _():
        st_ref[...] = blk

    @pl.when(pl.program_id(0) != 0)
    def _():
        st_ref[...] = st_ref[...] + blk


def _edge_stats(w1, g3, g4):
    return pl.pallas_call(
        _stats_body,
        grid=(E2 // EB2,),
        in_specs=[
            pl.BlockSpec((EB2, 2 * U), lambda i: (i, 0)),
            pl.BlockSpec((EB2, 2 * U), lambda i: (i, 0)),
            pl.BlockSpec((EB2, 2 * U), lambda i: (i, 0)),
        ],
        out_specs=[
            pl.BlockSpec((EB2, 2 * U), lambda i: (i, 0)),
            pl.BlockSpec((8, 2 * U), lambda i: (0, 0)),
        ],
        out_shape=[
            jax.ShapeDtypeStruct((E2, 2 * U), f32),
            jax.ShapeDtypeStruct((8, 2 * U), f32),
        ],
    )(w1, g3, g4)


def _node_stats_body(x1_ref, aggt_ref, o_ref, st_ref):
    p = aggt_ref[...]
    a = jnp.maximum(p[:U], p[U:])
    a = jnp.where(a == -jnp.inf, 0.0, a)
    t = x1_ref[...] + lax.dot_general(a, _eye(U),
                                      (((0,), (0,)), ((), ())),
                                      preferred_element_type=f32)
    o_ref[...] = t
    s = jnp.sum(t, axis=0, keepdims=True)
    q = jnp.sum(t * t, axis=0, keepdims=True)
    blk = jnp.concatenate([s, q, jnp.zeros((6, U), f32)], axis=0)

    @pl.when(pl.program_id(0) == 0)
    def _():
        st_ref[...] = blk

    @pl.when(pl.program_id(0) != 0)
    def _():
        st_ref[...] = st_ref[...] + blk


def _node_stats(x1, aggt):
    return pl.pallas_call(
        _node_stats_body,
        grid=(NP // NB,),
        in_specs=[
            pl.BlockSpec((NB, U), lambda i: (i, 0)),
            pl.BlockSpec((2 * U, NB), lambda i: (0, i)),
        ],
        out_specs=[
            pl.BlockSpec((NB, U), lambda i: (i, 0)),
            pl.BlockSpec((8, U), lambda i: (0, 0)),
        ],
        out_shape=[
            jax.ShapeDtypeStruct((NP, U), f32),
            jax.ShapeDtypeStruct((8, U), f32),
        ],
    )(x1, aggt)


def _fin_edge_body(x0_ref, t_ref, st_ref, g_ref, b_ref, o_ref):
    st = st_ref[...]
    s64 = st[0:1, :U] + st[0:1, U:]
    q64 = st[1:2, :U] + st[1:2, U:]
    mean = s64 / E
    var = q64 / E - mean * mean
    istd = lax.rsqrt(var + EPS)
    mean2 = jnp.concatenate([mean, mean], axis=1)
    istd2 = jnp.concatenate([istd, istd], axis=1)
    t = t_ref[...]
    bn = (t - mean2) * istd2 * g_ref[...] + b_ref[...]
    o_ref[...] = x0_ref[...] + _silu(bn)


def _fin_edge(w0, t, st, gam2, bet2):
    return pl.pallas_call(
        _fin_edge_body,
        grid=(E2 // EB2,),
        in_specs=[
            pl.BlockSpec((EB2, 2 * U), lambda i: (i, 0)),
            pl.BlockSpec((EB2, 2 * U), lambda i: (i, 0)),
            pl.BlockSpec((8, 2 * U), lambda i: (0, 0)),
            pl.BlockSpec((1, 2 * U), lambda i: (0, 0)),
            pl.BlockSpec((1, 2 * U), lambda i: (0, 0)),
        ],
        out_specs=pl.BlockSpec((EB2, 2 * U), lambda i: (i, 0)),
        out_shape=jax.ShapeDtypeStruct((E2, 2 * U), f32),
    )(w0, t, st, gam2, bet2)


def _fin_body(count, x0_ref, t_ref, st_ref, g_ref, b_ref, o_ref):
    st = st_ref[...]
    mean = st[0:1, :] / count
    var = st[1:2, :] / count - mean * mean
    istd = lax.rsqrt(var + EPS)
    t = t_ref[...]
    bn = (t - mean) * istd * g_ref[...] + b_ref[...]
    o_ref[...] = x0_ref[...] + _silu(bn)


def _finalize(x0, t, st, gam, bet, rows, blk):
    return pl.pallas_call(
        functools.partial(_fin_body, float(E if rows == E else N)),
        grid=(rows // blk,),
        in_specs=[
            pl.BlockSpec((blk, U), lambda i: (i, 0)),
            pl.BlockSpec((blk, U), lambda i: (i, 0)),
            pl.BlockSpec((8, U), lambda i: (0, 0)),
            pl.BlockSpec((1, U), lambda i: (0, 0)),
            pl.BlockSpec((1, U), lambda i: (0, 0)),
        ],
        out_specs=pl.BlockSpec((blk, U), lambda i: (i, 0)),
        out_shape=jax.ShapeDtypeStruct((rows, U), f32),
    )(x0, t, st, gam, bet)


_MESH = None


def _mesh():
    global _MESH
    if _MESH is None:
        _MESH = plsc.VectorSubcoreMesh(core_axis_name="c",
                                       subcore_axis_name="s")
    return _MESH


def _gather_add(x3, x4, src2d, dst2d):
    @functools.partial(
        pl.kernel,
        out_type=[jax.ShapeDtypeStruct((E, U), f32),
                  jax.ShapeDtypeStruct((E, U), f32)],
        mesh=_mesh(),
        compiler_params=_sc_params(tc_tiling=False),
        scratch_types=[pltpu.SemaphoreType.DMA, pltpu.SemaphoreType.DMA],
    )
    def k(x3_hbm, x4_hbm, src_hbm, dst_hbm, g3_hbm, g4_hbm, sem3, sem4):
        def body(s_v, d_v, o3_v, o4_v):
            c3 = pltpu.async_copy(x3_hbm.at[s_v.at[0]], o3_v, sem3)
            c4 = pltpu.async_copy(x4_hbm.at[d_v.at[0]], o4_v, sem4)
            c3.wait()
            c4.wait()

        pltpu.emit_pipeline(
            body,
            grid=(E // W_GATH,),
            in_specs=[
                pl.BlockSpec((1, W_GATH), lambda i: (0, i)),
                pl.BlockSpec((1, W_GATH), lambda i: (0, i)),
            ],
            out_specs=[pl.BlockSpec((W_GATH, U), lambda i: (i, 0)),
                       pl.BlockSpec((W_GATH, U), lambda i: (i, 0))],
            core_axis_name=("c", "s"),
            dimension_semantics=(pltpu.PARALLEL,),
        )(src_hbm, dst_hbm, g3_hbm, g4_hbm)

    return k(x3, x4, src2d, dst2d)


def _sc_params(tc_tiling=True):
    cp = pltpu.CompilerParams()
    if "needs_layout_passes" in pltpu.CompilerParams.__dataclass_fields__:
        cp = dataclasses.replace(cp, needs_layout_passes=False)
    if not tc_tiling:
        cp = dataclasses.replace(cp, use_tc_tiling_on_sc=False)
    return cp


def _segmax(x2t, w2t2, src_flat, dst_flat):
    NCH = E2 // SEG_CH
    GR = SEG_CH // 16

    @functools.partial(
        pl.kernel,
        out_type=jax.ShapeDtypeStruct((2 * U, NP), f32),
        mesh=_mesh(),
        compiler_params=_sc_params(),
        scratch_types=[
            pltpu.VMEM((4, NP), f32),
            pltpu.VMEM((4, NP), f32),
            pltpu.VMEM((4, SEG_CH), f32),
            pltpu.VMEM((4, SEG_CH), f32),
            pltpu.VMEM((2 * SEG_CH,), i32),
            pltpu.VMEM((2 * SEG_CH,), i32),
            pltpu.VMEM((2 * SEG_CH,), i32),
            pltpu.VMEM((2 * SEG_CH,), i32),
            pltpu.SemaphoreType.DMA,
            pltpu.SemaphoreType.DMA,
            pltpu.SemaphoreType.DMA,
        ],
    )
    def k(x2t_hbm, w2t_hbm, src_hbm, dst_hbm, agg_hbm,
          x2c, acc, w2a, w2b, sa, sb, da, db, semA, semB, semC):
        cid = lax.axis_index("c")
        sid = lax.axis_index("s")
        wid = sid * 2 + cid
        shard = wid & 1
        c0 = (wid // 2) * 4
        w2row = shard * U + c0

        pltpu.async_copy(x2t_hbm.at[pl.ds(c0, 4)], x2c, semC).wait()

        neg = jnp.full((16,), -jnp.inf, f32)

        @pl.loop(0, NP // 16)
        def _(i):
            for r in range(4):
                acc[r, pl.ds(i * 16, 16)] = neg

        def start(ch, w2buf, sbuf, dbuf, sem):
            e0 = ch * SEG_CH
            pltpu.async_copy(
                w2t_hbm.at[pl.ds(w2row, 4), pl.ds(e0, SEG_CH)], w2buf, sem)
            pltpu.async_copy(src_hbm.at[pl.ds(2 * e0, 2 * SEG_CH)], sbuf,
                             sem)
            pltpu.async_copy(dst_hbm.at[pl.ds(2 * e0, 2 * SEG_CH)], dbuf,
                             sem)

        def wait(w2buf, sbuf, dbuf, sem):
            pltpu.make_async_copy(
                w2t_hbm.at[pl.ds(0, 4), pl.ds(0, SEG_CH)], w2buf, sem).wait()
            pltpu.make_async_copy(src_hbm.at[pl.ds(0, 2 * SEG_CH)], sbuf,
                                  sem).wait()
            pltpu.make_async_copy(dst_hbm.at[pl.ds(0, 2 * SEG_CH)], dbuf,
                                  sem).wait()

        lane = lax.iota(i32, 16)
        lane2 = lane * 2 + shard
        ridx = [jnp.full((16,), r, i32) for r in range(4)]

        def process(w2buf, sbuf, dbuf):
            def group(b):
                pi = lane2 + 2 * b
                s = plsc.load_gather(sbuf, [pi])
                d = plsc.load_gather(dbuf, [pi])
                mv = [w2buf[r, pl.ds(b, 16)] *
                      plsc.load_gather(x2c, [ridx[r], d]) for r in range(4)]

                ss, perm = plsc.sort_key_val(s, lane)
                mv = [_vtake(m, perm) for m in mv]

                for dd in (1, 2, 4, 8):
                    idx = jnp.maximum(lane - dd, 0)
                    ks = _vtake(ss, idx)
                    take = jnp.logical_and(ks == ss, lane >= dd)
                    mv = [jnp.where(take, jnp.maximum(m, _vtake(m, idx)), m)
                          for m in mv]

                nxt = _vtake(ss, jnp.minimum(lane + 1, 15))
                last = jnp.logical_or(lane == 15, ss != nxt)

                for r in range(4):
                    a = plsc.load_gather(acc, [ridx[r], ss])
                    plsc.store_scatter(acc, [ridx[r], ss],
                                       jnp.maximum(a, mv[r]), mask=last)

            @pl.loop(0, GR)
            def _(g):
                group(g * 16)

        start(0, w2a, sa, da, semA)
        start(1, w2b, sb, db, semB)

        @pl.loop(0, NCH - 1, step=2)
        def _(ch):
            wait(w2a, sa, da, semA)
            process(w2a, sa, da)

            @pl.when(ch + 2 < NCH)
            def _():
                start(ch + 2, w2a, sa, da, semA)

            wait(w2b, sb, db, semB)
            process(w2b, sb, db)

            @pl.when(ch + 3 < NCH)
            def _():
                start(ch + 3, w2b, sb, db, semB)

        if NCH % 2 == 1:
            wait(w2a, sa, da, semA)
            process(w2a, sa, da)

        pltpu.async_copy(acc, agg_hbm.at[pl.ds(shard * U + c0, 4)],
                         semC).wait()

    return k(x2t, w2t2, src_flat, dst_flat)


def kernel(x, edge_index, edge_attr, params):
    src = edge_index[0].astype(i32)
    dst = edge_index[1].astype(i32)
    src2d = src.reshape(1, E)
    dst2d = dst.reshape(1, E)

    w0v, b0v = params['v_lin0']
    w0e, b0e = params['e_lin0']

    xp = jnp.pad(x, ((0, NP - N), (0, 0)))
    xc = _pre_x(xp, w0v, b0v.reshape(1, U))
    wc = _pre_w(edge_attr.reshape(E2, 4), w0e.T, b0e.reshape(1, U))

    for i in range(DEPTH):
        w1v, b1v = params['v_lins1'][i]
        w2v, b2v = params['v_lins2'][i]
        w3v, b3v = params['v_lins3'][i]
        w4v, b4v = params['v_lins4'][i]
        wev, bev = params['e_lins0'][i]
        gn, bn_ = params['v_bns'][i]
        ge, be_ = params['e_bns'][i]

        z = jnp.zeros((U, U), f32)
        wbd = jnp.block([[wev.T, z], [z, wev.T]])
        bev2 = jnp.tile(bev, 2).reshape(1, 2 * U)

        x1, x3, x4, x2t = _node_mm(
            xc, w1v, b1v.reshape(1, U), w2v, b2v.reshape(U, 1),
            w3v, b3v.reshape(1, U), w4v, b4v.reshape(1, U))
        w1, w2t2 = _edge_mm(wc, wbd, bev2)
        g3, g4 = _gather_add(x3, x4, src2d, dst2d)
        aggt = _segmax(x2t, w2t2, src, dst)
        te, ste = _edge_stats(w1, g3.reshape(E2, 2 * U),
                              g4.reshape(E2, 2 * U))
        tn, stn = _node_stats(x1, aggt)
        xc = _finalize(xc, tn, stn, gn.reshape(1, U), bn_.reshape(1, U),
                       NP, NB)
        wc = _fin_edge(wc, te, ste, jnp.tile(ge, 2).reshape(1, 2 * U),
                       jnp.tile(be_, 2).reshape(1, 2 * U))

    return xc[:N], wc.reshape(E, U)


# speedup vs baseline: 3.1024x; 1.0050x over previous
import dataclasses
import functools

import jax
import jax.numpy as jnp
from jax import lax
from jax.experimental import pallas as pl
from jax.experimental.pallas import tpu as pltpu
from jax.experimental.pallas import tpu_sc as plsc

N = 10000
NP = 10240
E = 320000
E2 = E // 2
EB2 = 1280
U = 64
F = 128
DEPTH = 3
NB = 2048
EB = 2560
W_GATH = 256
SEG_CH = 1280
EPS = 1e-5

f32 = jnp.float32
i32 = jnp.int32


_GDN = lax.GatherDimensionNumbers(offset_dims=(), collapsed_slice_dims=(0,),
                                  start_index_map=(0,))


def _vtake(v, idx):
    return lax.gather(v, idx[:, None], _GDN, (1,),
                      mode=lax.GatherScatterMode.PROMISE_IN_BOUNDS)


def _eye(n):
    r = lax.broadcasted_iota(i32, (n, n), 0)
    c = lax.broadcasted_iota(i32, (n, n), 1)
    return jnp.where(r == c, 1.0, 0.0).astype(f32)


def _silu(v):
    return v * jax.nn.sigmoid(v)


def _dotT(a, b):
    return lax.dot_general(a, b, (((1,), (1,)), ((), ())),
                           preferred_element_type=f32)


def _pre_x_body(x_ref, w_ref, b_ref, o_ref):
    pid = pl.program_id(0)
    xb = x_ref[...]
    rows = pid * NB + lax.broadcasted_iota(i32, (NB, U), 0)
    v = _dotT(xb, w_ref[...]) + b_ref[...]
    o_ref[...] = jnp.where(rows < N, _silu(v), 0.0)


def _pre_x(xp, w0, b0r):
    return pl.pallas_call(
        _pre_x_body,
        grid=(NP // NB,),
        in_specs=[
            pl.BlockSpec((NB, F), lambda i: (i, 0)),
            pl.BlockSpec((U, F), lambda i: (0, 0)),
            pl.BlockSpec((1, U), lambda i: (0, 0)),
        ],
        out_specs=pl.BlockSpec((NB, U), lambda i: (i, 0)),
        out_shape=jax.ShapeDtypeStruct((NP, U), f32),
    )(xp, w0, b0r)


def _pre_w_body(ea_ref, wt_ref, b_ref, o_ref):
    ea = ea_ref[...]
    wt = wt_ref[...]
    b = b_ref[...]
    ve = ea[:, 0:1] * wt[0:1, :] + ea[:, 1:2] * wt[1:2, :] + b
    vo = ea[:, 2:3] * wt[0:1, :] + ea[:, 3:4] * wt[1:2, :] + b
    o_ref[...] = _silu(jnp.concatenate([ve, vo], axis=1))


def _pre_w(ea4, wet, ber):
    return pl.pallas_call(
        _pre_w_body,
        grid=(E2 // EB2,),
        in_specs=[
            pl.BlockSpec((EB2, 4), lambda i: (i, 0)),
            pl.BlockSpec((2, U), lambda i: (0, 0)),
            pl.BlockSpec((1, U), lambda i: (0, 0)),
        ],
        out_specs=pl.BlockSpec((EB2, 2 * U), lambda i: (i, 0)),
        out_shape=jax.ShapeDtypeStruct((E2, 2 * U), f32),
    )(ea4, wet, ber)


def _node_mm_body(x_ref, w1, b1, w2, b2t, w3, b3, w4, b4,
                  x1_ref, x3_ref, x4_ref, x2t_ref):
    pid = pl.program_id(0)
    xb = x_ref[...]
    rows = pid * NB + lax.broadcasted_iota(i32, (NB, U), 0)
    rmask = rows < N

    def lin(wr, br):
        return jnp.where(rmask, _dotT(xb, wr[...]) + br[...], 0.0)

    x1_ref[...] = lin(w1, b1)
    x3_ref[...] = lin(w3, b3)
    x4_ref[...] = lin(w4, b4)
    cols = pid * NB + lax.broadcasted_iota(i32, (U, NB), 1)
    x2t = _dotT(w2[...], xb) + b2t[...]
    x2t_ref[...] = jnp.where(cols < N, x2t, 0.0)


def _node_mm(x0, w1, b1, w2, b2t, w3, b3, w4, b4):
    full = lambda i: (0, 0)
    return pl.pallas_call(
        _node_mm_body,
        grid=(NP // NB,),
        in_specs=[
            pl.BlockSpec((NB, U), lambda i: (i, 0)),
            pl.BlockSpec((U, U), full), pl.BlockSpec((1, U), full),
            pl.BlockSpec((U, U), full), pl.BlockSpec((U, 1), full),
            pl.BlockSpec((U, U), full), pl.BlockSpec((1, U), full),
            pl.BlockSpec((U, U), full), pl.BlockSpec((1, U), full),
        ],
        out_specs=[
            pl.BlockSpec((NB, U), lambda i: (i, 0)),
            pl.BlockSpec((NB, U), lambda i: (i, 0)),
            pl.BlockSpec((NB, U), lambda i: (i, 0)),
            pl.BlockSpec((U, NB), lambda i: (0, i)),
        ],
        out_shape=[
            jax.ShapeDtypeStruct((NP, U), f32),
            jax.ShapeDtypeStruct((NP, U), f32),
            jax.ShapeDtypeStruct((NP, U), f32),
            jax.ShapeDtypeStruct((U, NP), f32),
        ],
    )(x0, w1, b1, w2, b2t, w3, b3, w4, b4)


def _edge_mm_body(w_ref, wbd, be, w1_ref, w2t_ref):
    wb = w_ref[...]
    w1_ref[...] = jnp.dot(wb, wbd[...],
                          preferred_element_type=f32) + be[...]
    wbt = _dotT(_eye(2 * U), wb)
    w2t_ref[...] = jax.nn.sigmoid(wbt)


def _edge_mm(w0, wbd, ber):
    full = lambda i: (0, 0)
    return pl.pallas_call(
        _edge_mm_body,
        grid=(E2 // EB2,),
        in_specs=[
            pl.BlockSpec((EB2, 2 * U), lambda i: (i, 0)),
            pl.BlockSpec((2 * U, 2 * U), full),
            pl.BlockSpec((1, 2 * U), full),
        ],
        out_specs=[
            pl.BlockSpec((EB2, 2 * U), lambda i: (i, 0)),
            pl.BlockSpec((2 * U, EB2), lambda i: (0, i)),
        ],
        out_shape=[
            jax.ShapeDtypeStruct((E2, 2 * U), f32),
            jax.ShapeDtypeStruct((2 * U, E2), f32),
        ],
    )(w0, wbd, ber)


def _stats_body(a_ref, b_ref, c_ref, o_ref, st_ref):
    t = a_ref[...] + b_ref[...] + c_ref[...]
    o_ref[...] = t
    s = jnp.sum(t, axis=0, keepdims=True)
    q = jnp.sum(t * t, axis=0, keepdims=True)
    blk = jnp.concatenate([s, q, jnp.zeros((6, 2 * U), f32)], axis=0)

    @pl.when(pl.program_id(0) == 0)
    def _():
        st_ref[...] = blk

    @pl.when(pl.program_id(0) != 0)
    def _():
        st_ref[...] = st_ref[...] + blk


def _edge_stats(w1, g3, g4):
    return pl.pallas_call(
        _stats_body,
        grid=(E2 // EB2,),
        in_specs=[
            pl.BlockSpec((EB2, 2 * U), lambda i: (i, 0)),
            pl.BlockSpec((EB2, 2 * U), lambda i: (i, 0)),
            pl.BlockSpec((EB2, 2 * U), lambda i: (i, 0)),
        ],
        out_specs=[
            pl.BlockSpec((EB2, 2 * U), lambda i: (i, 0)),
            pl.BlockSpec((8, 2 * U), lambda i: (0, 0)),
        ],
        out_shape=[
            jax.ShapeDtypeStruct((E2, 2 * U), f32),
            jax.ShapeDtypeStruct((8, 2 * U), f32),
        ],
    )(w1, g3, g4)


def _node_stats_body(x1_ref, aggt_ref, o_ref, st_ref):
    p = aggt_ref[...]
    a = jnp.maximum(p[:U], p[U:])
    a = jnp.where(a == -jnp.inf, 0.0, a)
    t = x1_ref[...] + lax.dot_general(a, _eye(U),
                                      (((0,), (0,)), ((), ())),
                                      preferred_element_type=f32)
    o_ref[...] = t
    s = jnp.sum(t, axis=0, keepdims=True)
    q = jnp.sum(t * t, axis=0, keepdims=True)
    blk = jnp.concatenate([s, q, jnp.zeros((6, U), f32)], axis=0)

    @pl.when(pl.program_id(0) == 0)
    def _():
        st_ref[...] = blk

    @pl.when(pl.program_id(0) != 0)
    def _():
        st_ref[...] = st_ref[...] + blk


def _node_stats(x1, aggt):
    return pl.pallas_call(
        _node_stats_body,
        grid=(NP // NB,),
        in_specs=[
            pl.BlockSpec((NB, U), lambda i: (i, 0)),
            pl.BlockSpec((2 * U, NB), lambda i: (0, i)),
        ],
        out_specs=[
            pl.BlockSpec((NB, U), lambda i: (i, 0)),
            pl.BlockSpec((8, U), lambda i: (0, 0)),
        ],
        out_shape=[
            jax.ShapeDtypeStruct((NP, U), f32),
            jax.ShapeDtypeStruct((8, U), f32),
        ],
    )(x1, aggt)


def _fin_edge_body(x0_ref, t_ref, st_ref, g_ref, b_ref, o_ref):
    st = st_ref[...]
    s64 = st[0:1, :U] + st[0:1, U:]
    q64 = st[1:2, :U] + st[1:2, U:]
    mean = s64 / E
    var = q64 / E - mean * mean
    istd = lax.rsqrt(var + EPS)
    mean2 = jnp.concatenate([mean, mean], axis=1)
    istd2 = jnp.concatenate([istd, istd], axis=1)
    t = t_ref[...]
    bn = (t - mean2) * istd2 * g_ref[...] + b_ref[...]
    o_ref[...] = x0_ref[...] + _silu(bn)


def _fin_edge(w0, t, st, gam2, bet2):
    return pl.pallas_call(
        _fin_edge_body,
        grid=(E2 // EB2,),
        in_specs=[
            pl.BlockSpec((EB2, 2 * U), lambda i: (i, 0)),
            pl.BlockSpec((EB2, 2 * U), lambda i: (i, 0)),
            pl.BlockSpec((8, 2 * U), lambda i: (0, 0)),
            pl.BlockSpec((1, 2 * U), lambda i: (0, 0)),
            pl.BlockSpec((1, 2 * U), lambda i: (0, 0)),
        ],
        out_specs=pl.BlockSpec((EB2, 2 * U), lambda i: (i, 0)),
        out_shape=jax.ShapeDtypeStruct((E2, 2 * U), f32),
    )(w0, t, st, gam2, bet2)


def _fin_body(count, x0_ref, t_ref, st_ref, g_ref, b_ref, o_ref):
    st = st_ref[...]
    mean = st[0:1, :] / count
    var = st[1:2, :] / count - mean * mean
    istd = lax.rsqrt(var + EPS)
    t = t_ref[...]
    bn = (t - mean) * istd * g_ref[...] + b_ref[...]
    o_ref[...] = x0_ref[...] + _silu(bn)


def _finalize(x0, t, st, gam, bet, rows, blk):
    return pl.pallas_call(
        functools.partial(_fin_body, float(E if rows == E else N)),
        grid=(rows // blk,),
        in_specs=[
            pl.BlockSpec((blk, U), lambda i: (i, 0)),
            pl.BlockSpec((blk, U), lambda i: (i, 0)),
            pl.BlockSpec((8, U), lambda i: (0, 0)),
            pl.BlockSpec((1, U), lambda i: (0, 0)),
            pl.BlockSpec((1, U), lambda i: (0, 0)),
        ],
        out_specs=pl.BlockSpec((blk, U), lambda i: (i, 0)),
        out_shape=jax.ShapeDtypeStruct((rows, U), f32),
    )(x0, t, st, gam, bet)


_MESH = None


def _mesh():
    global _MESH
    if _MESH is None:
        _MESH = plsc.VectorSubcoreMesh(core_axis_name="c",
                                       subcore_axis_name="s")
    return _MESH


def _gather_add(x3, x4, src2d, dst2d):
    @functools.partial(
        pl.kernel,
        out_type=[jax.ShapeDtypeStruct((E, U), f32),
                  jax.ShapeDtypeStruct((E, U), f32)],
        mesh=_mesh(),
        compiler_params=_sc_params(tc_tiling=False),
        scratch_types=[pltpu.SemaphoreType.DMA, pltpu.SemaphoreType.DMA],
    )
    def k(x3_hbm, x4_hbm, src_hbm, dst_hbm, g3_hbm, g4_hbm, sem3, sem4):
        def body(s_v, d_v, o3_v, o4_v):
            cs = []
            for h in (0, 128):
                sl = pl.ds(h, 128)
                cs.append(pltpu.async_copy(
                    x3_hbm.at[s_v.at[0, sl]], o3_v.at[sl], sem3))
                cs.append(pltpu.async_copy(
                    x4_hbm.at[d_v.at[0, sl]], o4_v.at[sl], sem4))
            for c in cs:
                c.wait()

        pltpu.emit_pipeline(
            body,
            grid=(E // W_GATH,),
            in_specs=[
                pl.BlockSpec((1, W_GATH), lambda i: (0, i)),
                pl.BlockSpec((1, W_GATH), lambda i: (0, i)),
            ],
            out_specs=[pl.BlockSpec((W_GATH, U), lambda i: (i, 0)),
                       pl.BlockSpec((W_GATH, U), lambda i: (i, 0))],
            core_axis_name=("c", "s"),
            dimension_semantics=(pltpu.PARALLEL,),
        )(src_hbm, dst_hbm, g3_hbm, g4_hbm)

    return k(x3, x4, src2d, dst2d)


def _sc_params(tc_tiling=True):
    cp = pltpu.CompilerParams()
    if "needs_layout_passes" in pltpu.CompilerParams.__dataclass_fields__:
        cp = dataclasses.replace(cp, needs_layout_passes=False)
    if not tc_tiling:
        cp = dataclasses.replace(cp, use_tc_tiling_on_sc=False)
    return cp


def _segmax(x2t, w2t2, src_flat, dst_flat):
    NCH = E2 // SEG_CH
    GR = SEG_CH // 16

    @functools.partial(
        pl.kernel,
        out_type=jax.ShapeDtypeStruct((2 * U, NP), f32),
        mesh=_mesh(),
        compiler_params=_sc_params(),
        scratch_types=[
            pltpu.VMEM((4, NP), f32),
            pltpu.VMEM((4, NP), f32),
            pltpu.VMEM((4, SEG_CH), f32),
            pltpu.VMEM((4, SEG_CH), f32),
            pltpu.VMEM((2 * SEG_CH,), i32),
            pltpu.VMEM((2 * SEG_CH,), i32),
            pltpu.VMEM((2 * SEG_CH,), i32),
            pltpu.VMEM((2 * SEG_CH,), i32),
            pltpu.SemaphoreType.DMA,
            pltpu.SemaphoreType.DMA,
            pltpu.SemaphoreType.DMA,
        ],
    )
    def k(x2t_hbm, w2t_hbm, src_hbm, dst_hbm, agg_hbm,
          x2c, acc, w2a, w2b, sa, sb, da, db, semA, semB, semC):
        cid = lax.axis_index("c")
        sid = lax.axis_index("s")
        wid = sid * 2 + cid
        shard = wid & 1
        c0 = (wid // 2) * 4
        w2row = shard * U + c0

        pltpu.async_copy(x2t_hbm.at[pl.ds(c0, 4)], x2c, semC).wait()

        neg = jnp.full((16,), -jnp.inf, f32)

        @pl.loop(0, NP // 16)
        def _(i):
            for r in range(4):
                acc[r, pl.ds(i * 16, 16)] = neg

        def start(ch, w2buf, sbuf, dbuf, sem):
            e0 = ch * SEG_CH
            pltpu.async_copy(
                w2t_hbm.at[pl.ds(w2row, 4), pl.ds(e0, SEG_CH)], w2buf, sem)
            pltpu.async_copy(src_hbm.at[pl.ds(2 * e0, 2 * SEG_CH)], sbuf,
                             sem)
            pltpu.async_copy(dst_hbm.at[pl.ds(2 * e0, 2 * SEG_CH)], dbuf,
                             sem)

        def wait(w2buf, sbuf, dbuf, sem):
            pltpu.make_async_copy(
                w2t_hbm.at[pl.ds(0, 4), pl.ds(0, SEG_CH)], w2buf, sem).wait()
            pltpu.make_async_copy(src_hbm.at[pl.ds(0, 2 * SEG_CH)], sbuf,
                                  sem).wait()
            pltpu.make_async_copy(dst_hbm.at[pl.ds(0, 2 * SEG_CH)], dbuf,
                                  sem).wait()

        lane = lax.iota(i32, 16)
        lane2 = lane * 2 + shard
        ridx = [jnp.full((16,), r, i32) for r in range(4)]

        def process(w2buf, sbuf, dbuf):
            def group(b):
                pi = lane2 + 2 * b
                s = plsc.load_gather(sbuf, [pi])
                d = plsc.load_gather(dbuf, [pi])
                mv = [w2buf[r, pl.ds(b, 16)] *
                      plsc.load_gather(x2c, [ridx[r], d]) for r in range(4)]

                ss, perm = plsc.sort_key_val(s, lane)
                mv = [_vtake(m, perm) for m in mv]

                for dd in (1, 2, 4, 8):
                    idx = jnp.maximum(lane - dd, 0)
                    ks = _vtake(ss, idx)
                    take = jnp.logical_and(ks == ss, lane >= dd)
                    mv = [jnp.where(take, jnp.maximum(m, _vtake(m, idx)), m)
                          for m in mv]

                nxt = _vtake(ss, jnp.minimum(lane + 1, 15))
                last = jnp.logical_or(lane == 15, ss != nxt)

                for r in range(4):
                    a = plsc.load_gather(acc, [ridx[r], ss])
                    plsc.store_scatter(acc, [ridx[r], ss],
                                       jnp.maximum(a, mv[r]), mask=last)

            @pl.loop(0, GR)
            def _(g):
                group(g * 16)

        start(0, w2a, sa, da, semA)
        start(1, w2b, sb, db, semB)

        @pl.loop(0, NCH - 1, step=2)
        def _(ch):
            wait(w2a, sa, da, semA)
            process(w2a, sa, da)

            @pl.when(ch + 2 < NCH)
            def _():
                start(ch + 2, w2a, sa, da, semA)

            wait(w2b, sb, db, semB)
            process(w2b, sb, db)

            @pl.when(ch + 3 < NCH)
            def _():
                start(ch + 3, w2b, sb, db, semB)

        if NCH % 2 == 1:
            wait(w2a, sa, da, semA)
            process(w2a, sa, da)

        pltpu.async_copy(acc, agg_hbm.at[pl.ds(shard * U + c0, 4)],
                         semC).wait()

    return k(x2t, w2t2, src_flat, dst_flat)


def kernel(x, edge_index, edge_attr, params):
    src = edge_index[0].astype(i32)
    dst = edge_index[1].astype(i32)
    src2d = src.reshape(1, E)
    dst2d = dst.reshape(1, E)

    w0v, b0v = params['v_lin0']
    w0e, b0e = params['e_lin0']

    xp = jnp.pad(x, ((0, NP - N), (0, 0)))
    xc = _pre_x(xp, w0v, b0v.reshape(1, U))
    wc = _pre_w(edge_attr.reshape(E2, 4), w0e.T, b0e.reshape(1, U))

    for i in range(DEPTH):
        w1v, b1v = params['v_lins1'][i]
        w2v, b2v = params['v_lins2'][i]
        w3v, b3v = params['v_lins3'][i]
        w4v, b4v = params['v_lins4'][i]
        wev, bev = params['e_lins0'][i]
        gn, bn_ = params['v_bns'][i]
        ge, be_ = params['e_bns'][i]

        z = jnp.zeros((U, U), f32)
        wbd = jnp.block([[wev.T, z], [z, wev.T]])
        bev2 = jnp.tile(bev, 2).reshape(1, 2 * U)

        x1, x3, x4, x2t = _node_mm(
            xc, w1v, b1v.reshape(1, U), w2v, b2v.reshape(U, 1),
            w3v, b3v.reshape(1, U), w4v, b4v.reshape(1, U))
        w1, w2t2 = _edge_mm(wc, wbd, bev2)
        g3, g4 = _gather_add(x3, x4, src2d, dst2d)
        aggt = _segmax(x2t, w2t2, src, dst)
        te, ste = _edge_stats(w1, g3.reshape(E2, 2 * U),
                              g4.reshape(E2, 2 * U))
        tn, stn = _node_stats(x1, aggt)
        xc = _finalize(xc, tn, stn, gn.reshape(1, U), bn_.reshape(1, U),
                       NP, NB)
        wc = _fin_edge(wc, te, ste, jnp.tile(ge, 2).reshape(1, 2 * U),
                       jnp.tile(be_, 2).reshape(1, 2 * U))

    return xc[:N], wc.reshape(E, U)
